# Initial kernel scaffold; baseline (speedup 1.0000x reference)
import jax
import jax.numpy as jnp
from jax.experimental import pallas as pl


def kernel(x, pos, pe, edge_index, msg_w1, msg_b1, msg_w2, msg_b2, msgp_w1, msgp_b1, msgp_w2, msgp_b2, upd_w1, upd_b1, upd_w2, upd_b2, updp_w1, updp_b1, updp_w2, updp_b2):
    raise NotImplementedError("write your pallas kernel here")


# speedup vs baseline: 3.5380x; 3.5380x over previous
import functools

import jax
import jax.numpy as jnp
from jax import lax
from jax.experimental import pallas as pl
from jax.experimental.pallas import tpu as pltpu
from jax.experimental.pallas import tpu_sc as plsc

N = 10000
E = 320000
H = 128
TREC_D = 2 * H
TSEND_D = H
PD = 8

NC = 2
NS = 16
NW = NC * NS

EPW = E // NW
GK = 128
GFULL = EPW // GK
GTAIL = EPW - GFULL * GK

EPT = E // NS
SK = 128
SFULL = EPT // SK
STAIL = EPT - SFULL * SK
NROWS = 624
NREM = N - NS * NROWS


def _sc_mesh():
    return plsc.VectorSubcoreMesh(
        core_axis_name="c", subcore_axis_name="s", num_cores=NC, num_subcores=NS
    )


def _tc_precompute(x, pe, msg_w1, msg_b1, msgp_w1, msgp_b1):
    NB = 2000

    def body(x_r, pe_r, w1_r, b1_r, wp1_r, bp1_r, trec_r, tsend_r):
        xv = x_r[...]
        pev = pe_r[...]
        g = xv @ w1_r[0:H, :] + pev @ w1_r[H:2 * H, :] + b1_r[...]
        dr = pev @ wp1_r[H:2 * H, :] + bp1_r[...]
        cs = pev @ wp1_r[0:H, :]
        trec_r[...] = jnp.concatenate([g, dr], axis=1)
        tsend_r[...] = cs

    return pl.pallas_call(
        body,
        grid=(N // NB,),
        in_specs=[
            pl.BlockSpec((NB, H), lambda i: (i, 0)),
            pl.BlockSpec((NB, H), lambda i: (i, 0)),
            pl.BlockSpec((2 * H + 1, H), lambda i: (0, 0)),
            pl.BlockSpec((1, H), lambda i: (0, 0)),
            pl.BlockSpec((2 * H + 1, H), lambda i: (0, 0)),
            pl.BlockSpec((1, H), lambda i: (0, 0)),
        ],
        out_specs=[
            pl.BlockSpec((NB, TREC_D), lambda i: (i, 0)),
            pl.BlockSpec((NB, TSEND_D), lambda i: (i, 0)),
        ],
        out_shape=(
            jax.ShapeDtypeStruct((N, TREC_D), jnp.float32),
            jax.ShapeDtypeStruct((N, TSEND_D), jnp.float32),
        ),
    )(x, pe, msg_w1, msg_b1, msgp_w1, msgp_b1)


def _sc_gather(trec, tsend, posx, posy, posz, rec, send):
    @functools.partial(
        pl.kernel,
        out_type=(
            jax.ShapeDtypeStruct((E, TREC_D), jnp.float32),
            jax.ShapeDtypeStruct((E, TSEND_D), jnp.float32),
            jax.ShapeDtypeStruct((E, PD), jnp.float32),
        ),
        mesh=_sc_mesh(),
        scratch_types=[
            pltpu.VMEM((GK,), jnp.int32),
            pltpu.VMEM((GK,), jnp.int32),
            pltpu.VMEM((GK, TREC_D), jnp.float32),
            pltpu.VMEM((GK, TSEND_D), jnp.float32),
            pltpu.VMEM((GK, PD), jnp.float32),
            pltpu.VMEM((GTAIL,), jnp.int32),
            pltpu.VMEM((GTAIL,), jnp.int32),
            pltpu.VMEM((GTAIL, PD), jnp.float32),
            pltpu.VMEM((N,), jnp.float32),
            pltpu.VMEM((N,), jnp.float32),
            pltpu.VMEM((N,), jnp.float32),
            pltpu.SemaphoreType.DMA,
            pltpu.SemaphoreType.DMA,
        ],
        compiler_params=pltpu.CompilerParams(needs_layout_passes=False),
    )
    def k(trec_h, tsend_h, posx_h, posy_h, posz_h, rec_h, send_h,
          grec_h, gsend_h, pose_h,
          idx_r, idx_s, buf_r, buf_s, pose_b, idxt_r, idxt_s, poset_b,
          posx_v, posy_v, posz_v, sem_r, sem_s):
        wid = lax.axis_index("s") * NC + lax.axis_index("c")
        base_w = wid * EPW
        pltpu.sync_copy(posx_h, posx_v)
        pltpu.sync_copy(posy_h, posy_v)
        pltpu.sync_copy(posz_h, posz_v)
        pos_tabs = (posx_v, posy_v, posz_v)

        zeros16 = jnp.zeros((16,), jnp.float32)
        iota16 = lax.iota(jnp.int32, 16)

        def pose_fill(idx_s_ref, idx_r_ref, pose_ref, ngroups):
            for j in range(ngroups):
                ids = iota16 + j * 16
                si = idx_s_ref[pl.ds(j * 16, 16)]
                ri = idx_r_ref[pl.ds(j * 16, 16)]
                for c in range(3):
                    vs = plsc.load_gather(pos_tabs[c], [si])
                    vr = plsc.load_gather(pos_tabs[c], [ri])
                    plsc.store_scatter(
                        pose_ref, [ids, jnp.full((16,), c, jnp.int32)], vs)
                    plsc.store_scatter(
                        pose_ref, [ids, jnp.full((16,), 3 + c, jnp.int32)], vr)
                for c in (6, 7):
                    plsc.store_scatter(
                        pose_ref, [ids, jnp.full((16,), c, jnp.int32)], zeros16)

        def chunk(i, carry):
            base = base_w + i * GK
            pltpu.sync_copy(rec_h.at[pl.ds(base, GK)], idx_r)
            pltpu.sync_copy(send_h.at[pl.ds(base, GK)], idx_s)
            c1 = pltpu.async_copy(trec_h.at[idx_r], buf_r, sem_r)
            c2 = pltpu.async_copy(tsend_h.at[idx_s], buf_s, sem_s)
            pose_fill(idx_s, idx_r, pose_b, GK // 16)
            c1.wait()
            c2.wait()
            pltpu.sync_copy(buf_r, grec_h.at[pl.ds(base, GK)])
            pltpu.sync_copy(buf_s, gsend_h.at[pl.ds(base, GK)])
            pltpu.sync_copy(pose_b, pose_h.at[pl.ds(base, GK)])
            return carry

        lax.fori_loop(0, GFULL, chunk, 0)

        base = base_w + GFULL * GK
        pltpu.sync_copy(rec_h.at[pl.ds(base, GTAIL)], idxt_r)
        pltpu.sync_copy(send_h.at[pl.ds(base, GTAIL)], idxt_s)
        c1 = pltpu.async_copy(trec_h.at[idxt_r], buf_r.at[pl.ds(0, GTAIL)], sem_r)
        c2 = pltpu.async_copy(tsend_h.at[idxt_s], buf_s.at[pl.ds(0, GTAIL)], sem_s)
        pose_fill(idxt_s, idxt_r, poset_b, GTAIL // 16)
        c1.wait()
        c2.wait()
        pltpu.sync_copy(buf_r.at[pl.ds(0, GTAIL)], grec_h.at[pl.ds(base, GTAIL)])
        pltpu.sync_copy(buf_s.at[pl.ds(0, GTAIL)], gsend_h.at[pl.ds(base, GTAIL)])
        pltpu.sync_copy(poset_b, pose_h.at[pl.ds(base, GTAIL)])

    return k(trec, tsend, posx, posy, posz, rec, send)


def _tc_edges(grec, gsend, pose, msg_w1, msgp_w1, msg_w2, msg_b2,
              msgp_w2, msgp_b2):
    B = 2560

    def body(grec_r, gsend_r, pose_r, w1_r, wp1_r, w2_r, b2_r, wp2_r, bp2_r,
             msg_r, msgp_r):
        g = grec_r[:, 0:H]
        dr = grec_r[:, H:2 * H]
        cs = gsend_r[...]
        ps = pose_r[:, 0:3]
        pr = pose_r[:, 3:6]
        d = ps - pr
        dist = jnp.sqrt(jnp.sum(d * d, axis=1, keepdims=True))
        w1d = w1_r[2 * H:2 * H + 1, :]
        wp1d = wp1_r[2 * H:2 * H + 1, :]
        h1 = jax.nn.silu(g + dist * w1d)
        msg_r[...] = jax.nn.silu(h1 @ w2_r[...] + b2_r[...])
        h1p = jnp.tanh(cs + dr + dist * wp1d)
        msgp_r[...] = jnp.tanh(h1p @ wp2_r[...] + bp2_r[...])

    return pl.pallas_call(
        body,
        grid=(E // B,),
        in_specs=[
            pl.BlockSpec((B, TREC_D), lambda i: (i, 0)),
            pl.BlockSpec((B, TSEND_D), lambda i: (i, 0)),
            pl.BlockSpec((B, PD), lambda i: (i, 0)),
            pl.BlockSpec((2 * H + 1, H), lambda i: (0, 0)),
            pl.BlockSpec((2 * H + 1, H), lambda i: (0, 0)),
            pl.BlockSpec((H, H), lambda i: (0, 0)),
            pl.BlockSpec((1, H), lambda i: (0, 0)),
            pl.BlockSpec((H, H), lambda i: (0, 0)),
            pl.BlockSpec((1, H), lambda i: (0, 0)),
        ],
        out_specs=[
            pl.BlockSpec((B, H), lambda i: (i, 0)),
            pl.BlockSpec((B, H), lambda i: (i, 0)),
        ],
        out_shape=(
            jax.ShapeDtypeStruct((E, H), jnp.float32),
            jax.ShapeDtypeStruct((E, H), jnp.float32),
        ),
    )(grec, gsend, pose, msg_w1, msgp_w1, msg_w2, msg_b2, msgp_w2, msgp_b2)


def _sc_scatter(msg, msgp, rec, zeros):
    @functools.partial(
        pl.kernel,
        out_type=(
            jax.ShapeDtypeStruct((N, H), jnp.float32),
            jax.ShapeDtypeStruct((N, H), jnp.float32),
        ),
        mesh=_sc_mesh(),
        scratch_types=[
            pltpu.VMEM_SHARED((N, H), jnp.float32),
            pltpu.VMEM((SK,), jnp.int32),
            pltpu.VMEM((SK, H), jnp.float32),
            pltpu.VMEM((STAIL,), jnp.int32),
            pltpu.VMEM((STAIL, H), jnp.float32),
        ],
    )
    def k(msg_h, msgp_h, rec_h, zeros_h, aggr_h, aggrp_h,
          acc_s, idx_v, mbuf, idxt_v, mbuft):
        cid = lax.axis_index("c")
        sid = lax.axis_index("s")
        pltpu.sync_copy(zeros_h.at[pl.ds(sid * NROWS, NROWS)],
                        acc_s.at[pl.ds(sid * NROWS, NROWS)])

        @pl.when(sid == NS - 1)
        def _():
            pltpu.sync_copy(zeros_h.at[pl.ds(NS * NROWS, NREM)],
                            acc_s.at[pl.ds(NS * NROWS, NREM)])

        plsc.subcore_barrier()

        def run(src_h):
            base_t = sid * EPT

            def chunk(i, carry):
                base = base_t + i * SK
                pltpu.sync_copy(rec_h.at[pl.ds(base, SK)], idx_v)
                pltpu.sync_copy(src_h.at[pl.ds(base, SK)], mbuf)
                pltpu.sync_copy(mbuf, acc_s.at[idx_v], add=True)
                return carry

            lax.fori_loop(0, SFULL, chunk, 0)
            base = base_t + SFULL * SK
            pltpu.sync_copy(rec_h.at[pl.ds(base, STAIL)], idxt_v)
            pltpu.sync_copy(src_h.at[pl.ds(base, STAIL)], mbuft)
            pltpu.sync_copy(mbuft, acc_s.at[idxt_v], add=True)

        @pl.when(cid == 0)
        def _():
            run(msg_h)

        @pl.when(cid == 1)
        def _():
            run(msgp_h)

        plsc.subcore_barrier()

        @pl.when(cid == 0)
        def _():
            pltpu.sync_copy(acc_s.at[pl.ds(sid * NROWS, NROWS)],
                            aggr_h.at[pl.ds(sid * NROWS, NROWS)])

        @pl.when(cid == 1)
        def _():
            pltpu.sync_copy(acc_s.at[pl.ds(sid * NROWS, NROWS)],
                            aggrp_h.at[pl.ds(sid * NROWS, NROWS)])

        @pl.when((sid == NS - 1) & (cid == 0))
        def _():
            pltpu.sync_copy(acc_s.at[pl.ds(NS * NROWS, NREM)],
                            aggr_h.at[pl.ds(NS * NROWS, NREM)])

        @pl.when((sid == NS - 1) & (cid == 1))
        def _():
            pltpu.sync_copy(acc_s.at[pl.ds(NS * NROWS, NREM)],
                            aggrp_h.at[pl.ds(NS * NROWS, NREM)])

    return k(msg, msgp, rec, zeros)


def _tc_update(x, pe, aggr, aggrp, u1, ub1, u2, ub2, p1, pb1, p2, pb2):
    NB = 2000

    def body(x_r, pe_r, a_r, ap_r, u1_r, ub1_r, u2_r, ub2_r,
             p1_r, pb1_r, p2_r, pb2_r, out_r, outp_r):
        xv = x_r[...]
        pev = pe_r[...]
        t = (xv @ u1_r[0:H, :] + pev @ u1_r[H:2 * H, :]
             + a_r[...] @ u1_r[2 * H:3 * H, :] + ub1_r[...])
        out_r[...] = jax.nn.silu(t) @ u2_r[...] + ub2_r[...]
        tp = pev @ p1_r[0:H, :] + ap_r[...] @ p1_r[H:2 * H, :] + pb1_r[...]
        outp_r[...] = jnp.tanh(jnp.tanh(tp) @ p2_r[...] + pb2_r[...])

    return pl.pallas_call(
        body,
        grid=(N // NB,),
        in_specs=[
            pl.BlockSpec((NB, H), lambda i: (i, 0)),
            pl.BlockSpec((NB, H), lambda i: (i, 0)),
            pl.BlockSpec((NB, H), lambda i: (i, 0)),
            pl.BlockSpec((NB, H), lambda i: (i, 0)),
            pl.BlockSpec((3 * H, H), lambda i: (0, 0)),
            pl.BlockSpec((1, H), lambda i: (0, 0)),
            pl.BlockSpec((H, H), lambda i: (0, 0)),
            pl.BlockSpec((1, H), lambda i: (0, 0)),
            pl.BlockSpec((2 * H, H), lambda i: (0, 0)),
            pl.BlockSpec((1, H), lambda i: (0, 0)),
            pl.BlockSpec((H, H), lambda i: (0, 0)),
            pl.BlockSpec((1, H), lambda i: (0, 0)),
        ],
        out_specs=[
            pl.BlockSpec((NB, H), lambda i: (i, 0)),
            pl.BlockSpec((NB, H), lambda i: (i, 0)),
        ],
        out_shape=(
            jax.ShapeDtypeStruct((N, H), jnp.float32),
            jax.ShapeDtypeStruct((N, H), jnp.float32),
        ),
    )(x, pe, aggr, aggrp, u1, ub1, u2, ub2, p1, pb1, p2, pb2)


def kernel(x, pos, pe, edge_index, msg_w1, msg_b1, msg_w2, msg_b2,
           msgp_w1, msgp_b1, msgp_w2, msgp_b2, upd_w1, upd_b1, upd_w2,
           upd_b2, updp_w1, updp_b1, updp_w2, updp_b2):
    send = edge_index[0]
    rec = edge_index[1]

    b1 = msg_b1.reshape(1, H)
    b2 = msg_b2.reshape(1, H)
    bp1 = msgp_b1.reshape(1, H)
    bp2 = msgp_b2.reshape(1, H)
    ub1 = upd_b1.reshape(1, H)
    ub2 = upd_b2.reshape(1, H)
    pb1 = updp_b1.reshape(1, H)
    pb2 = updp_b2.reshape(1, H)

    posx = pos[:, 0]
    posy = pos[:, 1]
    posz = pos[:, 2]

    trec, tsend = _tc_precompute(x, pe, msg_w1, b1, msgp_w1, bp1)
    grec, gsend, pose = _sc_gather(trec, tsend, posx, posy, posz, rec, send)
    msg, msgp = _tc_edges(grec, gsend, pose, msg_w1, msgp_w1, msg_w2, b2,
                          msgp_w2, bp2)
    zeros = jnp.zeros((N, H), jnp.float32)
    aggr, aggrp = _sc_scatter(msg, msgp, rec, zeros)
    return _tc_update(x, pe, aggr, aggrp, upd_w1, ub1, upd_w2, ub2,
                      updp_w1, pb1, updp_w2, pb2)


# speedup vs baseline: 4.2433x; 1.1994x over previous
import functools

import jax
import jax.numpy as jnp
from jax import lax
from jax.experimental import pallas as pl
from jax.experimental.pallas import tpu as pltpu
from jax.experimental.pallas import tpu_sc as plsc

N = 10000
E = 320000
H = 128
TREC_D = 2 * H
TSEND_D = H
PD = 8

NC = 2
NS = 16
NW = NC * NS

EPW = E // NW
GK = 96
GFULL = EPW // GK
GPAIRS = GFULL // 2
GTAIL = EPW - GFULL * GK

EPT = E // NS
SK = 128
SFULL = EPT // SK
SPAIRS = SFULL // 2
STAIL = EPT - SFULL * SK
NROWS = 624
NREM = N - NS * NROWS


def _sc_mesh():
    return plsc.VectorSubcoreMesh(
        core_axis_name="c", subcore_axis_name="s", num_cores=NC, num_subcores=NS
    )


def _tc_precompute(x, pe, msg_w1, msg_b1, msgp_w1, msgp_b1):
    NB = 2000

    def body(x_r, pe_r, w1_r, b1_r, wp1_r, bp1_r, trec_r, tsend_r):
        xv = x_r[...]
        pev = pe_r[...]
        g = xv @ w1_r[0:H, :] + pev @ w1_r[H:2 * H, :] + b1_r[...]
        dr = pev @ wp1_r[H:2 * H, :] + bp1_r[...]
        cs = pev @ wp1_r[0:H, :]
        trec_r[...] = jnp.concatenate([g, dr], axis=1)
        tsend_r[...] = cs

    return pl.pallas_call(
        body,
        grid=(N // NB,),
        in_specs=[
            pl.BlockSpec((NB, H), lambda i: (i, 0)),
            pl.BlockSpec((NB, H), lambda i: (i, 0)),
            pl.BlockSpec((2 * H + 1, H), lambda i: (0, 0)),
            pl.BlockSpec((1, H), lambda i: (0, 0)),
            pl.BlockSpec((2 * H + 1, H), lambda i: (0, 0)),
            pl.BlockSpec((1, H), lambda i: (0, 0)),
        ],
        out_specs=[
            pl.BlockSpec((NB, TREC_D), lambda i: (i, 0)),
            pl.BlockSpec((NB, TSEND_D), lambda i: (i, 0)),
        ],
        out_shape=(
            jax.ShapeDtypeStruct((N, TREC_D), jnp.float32),
            jax.ShapeDtypeStruct((N, TSEND_D), jnp.float32),
        ),
    )(x, pe, msg_w1, msg_b1, msgp_w1, msgp_b1)


def _sc_gather(trec, tsend, posx, posy, posz, rec, send):
    @functools.partial(
        pl.kernel,
        out_type=(
            jax.ShapeDtypeStruct((E, TREC_D), jnp.float32),
            jax.ShapeDtypeStruct((E, PD), jnp.float32),
        ),
        mesh=_sc_mesh(),
        scratch_types=[
            pltpu.VMEM((GK,), jnp.int32),
            pltpu.VMEM((GK,), jnp.int32),
            pltpu.VMEM((GK, TREC_D), jnp.float32),
            pltpu.VMEM((GK, TSEND_D), jnp.float32),
            pltpu.VMEM((GK, PD), jnp.float32),
            pltpu.VMEM((GK,), jnp.int32),
            pltpu.VMEM((GK,), jnp.int32),
            pltpu.VMEM((GK, TREC_D), jnp.float32),
            pltpu.VMEM((GK, TSEND_D), jnp.float32),
            pltpu.VMEM((GK, PD), jnp.float32),
            pltpu.VMEM((GTAIL,), jnp.int32),
            pltpu.VMEM((GTAIL,), jnp.int32),
            pltpu.VMEM((N,), jnp.float32),
            pltpu.VMEM((N,), jnp.float32),
            pltpu.VMEM((N,), jnp.float32),
            pltpu.SemaphoreType.DMA,
            pltpu.SemaphoreType.DMA,
            pltpu.SemaphoreType.DMA,
            pltpu.SemaphoreType.DMA,
        ],
        compiler_params=pltpu.CompilerParams(needs_layout_passes=False),
    )
    def k(trec_h, tsend_h, posx_h, posy_h, posz_h, rec_h, send_h,
          grec_h, pose_h,
          idx_r0, idx_s0, buf_r0, buf_s0, pose0,
          idx_r1, idx_s1, buf_r1, buf_s1, pose1,
          idxt_r, idxt_s,
          posx_v, posy_v, posz_v,
          gsem0, gsem1, wsem0, wsem1):
        wid = lax.axis_index("s") * NC + lax.axis_index("c")
        base_w = wid * EPW
        pltpu.sync_copy(posx_h, posx_v)
        pltpu.sync_copy(posy_h, posy_v)
        pltpu.sync_copy(posz_h, posz_v)
        pos_tabs = (posx_v, posy_v, posz_v)

        zeros16 = jnp.zeros((16,), jnp.float32)
        iota16 = lax.iota(jnp.int32, 16)

        def load_idx(c, idx_r, idx_s, n):
            base = base_w + c * GK
            pltpu.sync_copy(rec_h.at[pl.ds(base, n)], idx_r)
            pltpu.sync_copy(send_h.at[pl.ds(base, n)], idx_s)

        def start_gather(idx_r, idx_s, buf_r, buf_s, gsem):
            pltpu.async_copy(trec_h.at[idx_r], buf_r, gsem)
            pltpu.async_copy(tsend_h.at[idx_s], buf_s, gsem)

        def wait_gather(idx_r, idx_s, buf_r, buf_s, gsem):
            pltpu.make_async_copy(trec_h.at[idx_r], buf_r, gsem).wait()
            pltpu.make_async_copy(tsend_h.at[idx_s], buf_s, gsem).wait()

        def do_adds(buf_r, buf_s, nrows):
            def row(e, carry):
                for kk in range(H // 16):
                    v = buf_s[e, pl.ds(kk * 16, 16)]
                    plsc.addupdate(buf_r.at[e, pl.ds(H + kk * 16, 16)], v)
                return carry
            lax.fori_loop(0, nrows, row, 0)

        def pose_fill(idx_s_ref, idx_r_ref, pose_ref, ngroups):
            for j in range(ngroups):
                ids = iota16 + j * 16
                si = idx_s_ref[pl.ds(j * 16, 16)]
                ri = idx_r_ref[pl.ds(j * 16, 16)]
                for c in range(3):
                    vs = plsc.load_gather(pos_tabs[c], [si])
                    vr = plsc.load_gather(pos_tabs[c], [ri])
                    plsc.store_scatter(
                        pose_ref, [ids, jnp.full((16,), c, jnp.int32)], vs)
                    plsc.store_scatter(
                        pose_ref, [ids, jnp.full((16,), 3 + c, jnp.int32)], vr)
                for c in (6, 7):
                    plsc.store_scatter(
                        pose_ref, [ids, jnp.full((16,), c, jnp.int32)], zeros16)

        def start_writes(buf_r, pose_b, c, wsem):
            base = base_w + c * GK
            pltpu.async_copy(buf_r, grec_h.at[pl.ds(base, GK)], wsem)
            pltpu.async_copy(pose_b, pose_h.at[pl.ds(base, GK)], wsem)

        def wait_writes(buf_r, pose_b, c, wsem):
            base = base_w + c * GK
            pltpu.make_async_copy(buf_r, grec_h.at[pl.ds(base, GK)], wsem).wait()
            pltpu.make_async_copy(pose_b, pose_h.at[pl.ds(base, GK)], wsem).wait()

        load_idx(0, idx_r0, idx_s0, GK)
        start_gather(idx_r0, idx_s0, buf_r0, buf_s0, gsem0)

        def pair(i, carry):
            c0 = 2 * i
            c1 = c0 + 1
            load_idx(c1, idx_r1, idx_s1, GK)

            @pl.when(i > 0)
            def _():
                wait_writes(buf_r1, pose1, c1 - 2, wsem1)

            start_gather(idx_r1, idx_s1, buf_r1, buf_s1, gsem1)
            pose_fill(idx_s0, idx_r0, pose0, GK // 16)
            wait_gather(idx_r0, idx_s0, buf_r0, buf_s0, gsem0)
            do_adds(buf_r0, buf_s0, GK)
            start_writes(buf_r0, pose0, c0, wsem0)

            @pl.when(i < GPAIRS - 1)
            def _():
                load_idx(c0 + 2, idx_r0, idx_s0, GK)
                wait_writes(buf_r0, pose0, c0, wsem0)
                start_gather(idx_r0, idx_s0, buf_r0, buf_s0, gsem0)

            pose_fill(idx_s1, idx_r1, pose1, GK // 16)
            wait_gather(idx_r1, idx_s1, buf_r1, buf_s1, gsem1)
            do_adds(buf_r1, buf_s1, GK)
            start_writes(buf_r1, pose1, c1, wsem1)
            return carry

        lax.fori_loop(0, GPAIRS, pair, 0)
        wait_writes(buf_r0, pose0, GFULL - 2, wsem0)
        wait_writes(buf_r1, pose1, GFULL - 1, wsem1)

        base = base_w + GFULL * GK
        pltpu.sync_copy(rec_h.at[pl.ds(base, GTAIL)], idxt_r)
        pltpu.sync_copy(send_h.at[pl.ds(base, GTAIL)], idxt_s)
        c1 = pltpu.async_copy(trec_h.at[idxt_r],
                              buf_r0.at[pl.ds(0, GTAIL)], gsem0)
        c2 = pltpu.async_copy(tsend_h.at[idxt_s],
                              buf_s0.at[pl.ds(0, GTAIL)], gsem0)
        pose_fill(idxt_s, idxt_r, pose0, GTAIL // 16)
        c1.wait()
        c2.wait()
        do_adds(buf_r0, buf_s0, GTAIL)
        pltpu.sync_copy(buf_r0.at[pl.ds(0, GTAIL)],
                        grec_h.at[pl.ds(base, GTAIL)])
        pltpu.sync_copy(pose0.at[pl.ds(0, GTAIL)],
                        pose_h.at[pl.ds(base, GTAIL)])

    return k(trec, tsend, posx, posy, posz, rec, send)


def _tc_edges(grec, pose, msg_w1, msgp_w1, msg_w2, msg_b2, msgp_w2, msgp_b2):
    B = 2560

    def body(grec_r, pose_r, w1_r, wp1_r, w2_r, b2_r, wp2_r, bp2_r,
             msg_r, msgp_r):
        g = grec_r[:, 0:H]
        s2 = grec_r[:, H:2 * H]
        ps = pose_r[:, 0:3]
        pr = pose_r[:, 3:6]
        d = ps - pr
        dist = jnp.sqrt(jnp.sum(d * d, axis=1, keepdims=True))
        w1d = w1_r[2 * H:2 * H + 1, :]
        wp1d = wp1_r[2 * H:2 * H + 1, :]
        h1 = jax.nn.silu(g + dist * w1d)
        msg_r[...] = jax.nn.silu(h1 @ w2_r[...] + b2_r[...])
        h1p = jnp.tanh(s2 + dist * wp1d)
        msgp_r[...] = jnp.tanh(h1p @ wp2_r[...] + bp2_r[...])

    return pl.pallas_call(
        body,
        grid=(E // B,),
        in_specs=[
            pl.BlockSpec((B, TREC_D), lambda i: (i, 0)),
            pl.BlockSpec((B, PD), lambda i: (i, 0)),
            pl.BlockSpec((2 * H + 1, H), lambda i: (0, 0)),
            pl.BlockSpec((2 * H + 1, H), lambda i: (0, 0)),
            pl.BlockSpec((H, H), lambda i: (0, 0)),
            pl.BlockSpec((1, H), lambda i: (0, 0)),
            pl.BlockSpec((H, H), lambda i: (0, 0)),
            pl.BlockSpec((1, H), lambda i: (0, 0)),
        ],
        out_specs=[
            pl.BlockSpec((B, H), lambda i: (i, 0)),
            pl.BlockSpec((B, H), lambda i: (i, 0)),
        ],
        out_shape=(
            jax.ShapeDtypeStruct((E, H), jnp.float32),
            jax.ShapeDtypeStruct((E, H), jnp.float32),
        ),
    )(grec, pose, msg_w1, msgp_w1, msg_w2, msg_b2, msgp_w2, msgp_b2)


def _sc_scatter(msg, msgp, rec, zeros):
    @functools.partial(
        pl.kernel,
        out_type=(
            jax.ShapeDtypeStruct((N, H), jnp.float32),
            jax.ShapeDtypeStruct((N, H), jnp.float32),
        ),
        mesh=_sc_mesh(),
        scratch_types=[
            pltpu.VMEM_SHARED((N, H), jnp.float32),
            pltpu.VMEM((SK,), jnp.int32),
            pltpu.VMEM((SK, H), jnp.float32),
            pltpu.VMEM((SK,), jnp.int32),
            pltpu.VMEM((SK, H), jnp.float32),
            pltpu.VMEM((STAIL,), jnp.int32),
            pltpu.VMEM((STAIL, H), jnp.float32),
            pltpu.SemaphoreType.DMA,
            pltpu.SemaphoreType.DMA,
        ],
    )
    def k(msg_h, msgp_h, rec_h, zeros_h, aggr_h, aggrp_h,
          acc_s, idx0, mb0, idx1, mb1, idxt, mbt, lsem0, lsem1):
        cid = lax.axis_index("c")
        sid = lax.axis_index("s")
        pltpu.sync_copy(zeros_h.at[pl.ds(sid * NROWS, NROWS)],
                        acc_s.at[pl.ds(sid * NROWS, NROWS)])

        @pl.when(sid == NS - 1)
        def _():
            pltpu.sync_copy(zeros_h.at[pl.ds(NS * NROWS, NREM)],
                            acc_s.at[pl.ds(NS * NROWS, NREM)])

        plsc.subcore_barrier()

        def run(src_h):
            base_t = sid * EPT

            def load(c, idx, mb, lsem):
                base = base_t + c * SK
                pltpu.sync_copy(rec_h.at[pl.ds(base, SK)], idx)
                pltpu.async_copy(src_h.at[pl.ds(base, SK)], mb, lsem)

            def wait_load(c, mb, lsem):
                base = base_t + c * SK
                pltpu.make_async_copy(src_h.at[pl.ds(base, SK)], mb,
                                      lsem).wait()

            load(0, idx0, mb0, lsem0)

            def pair(i, carry):
                c0 = 2 * i
                c1 = c0 + 1
                load(c1, idx1, mb1, lsem1)
                wait_load(c0, mb0, lsem0)
                pltpu.sync_copy(mb0, acc_s.at[idx0], add=True)

                @pl.when(i < SPAIRS - 1)
                def _():
                    load(c0 + 2, idx0, mb0, lsem0)

                wait_load(c1, mb1, lsem1)
                pltpu.sync_copy(mb1, acc_s.at[idx1], add=True)
                return carry

            lax.fori_loop(0, SPAIRS, pair, 0)
            base = base_t + SFULL * SK
            pltpu.sync_copy(rec_h.at[pl.ds(base, STAIL)], idxt)
            pltpu.sync_copy(src_h.at[pl.ds(base, STAIL)], mbt)
            pltpu.sync_copy(mbt, acc_s.at[idxt], add=True)

        @pl.when(cid == 0)
        def _():
            run(msg_h)

        @pl.when(cid == 1)
        def _():
            run(msgp_h)

        plsc.subcore_barrier()

        @pl.when(cid == 0)
        def _():
            pltpu.sync_copy(acc_s.at[pl.ds(sid * NROWS, NROWS)],
                            aggr_h.at[pl.ds(sid * NROWS, NROWS)])

        @pl.when(cid == 1)
        def _():
            pltpu.sync_copy(acc_s.at[pl.ds(sid * NROWS, NROWS)],
                            aggrp_h.at[pl.ds(sid * NROWS, NROWS)])

        @pl.when((sid == NS - 1) & (cid == 0))
        def _():
            pltpu.sync_copy(acc_s.at[pl.ds(NS * NROWS, NREM)],
                            aggr_h.at[pl.ds(NS * NROWS, NREM)])

        @pl.when((sid == NS - 1) & (cid == 1))
        def _():
            pltpu.sync_copy(acc_s.at[pl.ds(NS * NROWS, NREM)],
                            aggrp_h.at[pl.ds(NS * NROWS, NREM)])

    return k(msg, msgp, rec, zeros)


def _tc_update(x, pe, aggr, aggrp, u1, ub1, u2, ub2, p1, pb1, p2, pb2):
    NB = 2000

    def body(x_r, pe_r, a_r, ap_r, u1_r, ub1_r, u2_r, ub2_r,
             p1_r, pb1_r, p2_r, pb2_r, out_r, outp_r):
        xv = x_r[...]
        pev = pe_r[...]
        t = (xv @ u1_r[0:H, :] + pev @ u1_r[H:2 * H, :]
             + a_r[...] @ u1_r[2 * H:3 * H, :] + ub1_r[...])
        out_r[...] = jax.nn.silu(t) @ u2_r[...] + ub2_r[...]
        tp = pev @ p1_r[0:H, :] + ap_r[...] @ p1_r[H:2 * H, :] + pb1_r[...]
        outp_r[...] = jnp.tanh(jnp.tanh(tp) @ p2_r[...] + pb2_r[...])

    return pl.pallas_call(
        body,
        grid=(N // NB,),
        in_specs=[
            pl.BlockSpec((NB, H), lambda i: (i, 0)),
            pl.BlockSpec((NB, H), lambda i: (i, 0)),
            pl.BlockSpec((NB, H), lambda i: (i, 0)),
            pl.BlockSpec((NB, H), lambda i: (i, 0)),
            pl.BlockSpec((3 * H, H), lambda i: (0, 0)),
            pl.BlockSpec((1, H), lambda i: (0, 0)),
            pl.BlockSpec((H, H), lambda i: (0, 0)),
            pl.BlockSpec((1, H), lambda i: (0, 0)),
            pl.BlockSpec((2 * H, H), lambda i: (0, 0)),
            pl.BlockSpec((1, H), lambda i: (0, 0)),
            pl.BlockSpec((H, H), lambda i: (0, 0)),
            pl.BlockSpec((1, H), lambda i: (0, 0)),
        ],
        out_specs=[
            pl.BlockSpec((NB, H), lambda i: (i, 0)),
            pl.BlockSpec((NB, H), lambda i: (i, 0)),
        ],
        out_shape=(
            jax.ShapeDtypeStruct((N, H), jnp.float32),
            jax.ShapeDtypeStruct((N, H), jnp.float32),
        ),
    )(x, pe, aggr, aggrp, u1, ub1, u2, ub2, p1, pb1, p2, pb2)


def kernel(x, pos, pe, edge_index, msg_w1, msg_b1, msg_w2, msg_b2,
           msgp_w1, msgp_b1, msgp_w2, msgp_b2, upd_w1, upd_b1, upd_w2,
           upd_b2, updp_w1, updp_b1, updp_w2, updp_b2):
    send = edge_index[0]
    rec = edge_index[1]

    b1 = msg_b1.reshape(1, H)
    b2 = msg_b2.reshape(1, H)
    bp1 = msgp_b1.reshape(1, H)
    bp2 = msgp_b2.reshape(1, H)
    ub1 = upd_b1.reshape(1, H)
    ub2 = upd_b2.reshape(1, H)
    pb1 = updp_b1.reshape(1, H)
    pb2 = updp_b2.reshape(1, H)

    posx = pos[:, 0]
    posy = pos[:, 1]
    posz = pos[:, 2]

    trec, tsend = _tc_precompute(x, pe, msg_w1, b1, msgp_w1, bp1)
    grec, pose = _sc_gather(trec, tsend, posx, posy, posz, rec, send)
    msg, msgp = _tc_edges(grec, pose, msg_w1, msgp_w1, msg_w2, b2,
                          msgp_w2, bp2)
    zeros = jnp.zeros((N, H), jnp.float32)
    aggr, aggrp = _sc_scatter(msg, msgp, rec, zeros)
    return _tc_update(x, pe, aggr, aggrp, upd_w1, ub1, upd_w2, ub2,
                      updp_w1, pb1, updp_w2, pb2)


# speedup vs baseline: 4.7419x; 1.1175x over previous
import functools

import jax
import jax.numpy as jnp
from jax import lax
from jax.experimental import pallas as pl
from jax.experimental.pallas import tpu as pltpu
from jax.experimental.pallas import tpu_sc as plsc

N = 10000
E = 320000
H = 128
TREC_D = 2 * H
TSEND_D = H
PD = 8

NC = 2
NS = 16
NW = NC * NS

NSLAB = 5
SLAB = E // NSLAB

EPW = SLAB // NW
GK = 96
GFULL = EPW // GK
GPAIRS = GFULL // 2
GTAIL = EPW - GFULL * GK

EPTS = SLAB // NS
SK = 128
SFULL = EPTS // SK
SPAIRS = SFULL // 2
STAIL = EPTS - SFULL * SK
NROWS = 624
NREM = N - NS * NROWS


def _sc_mesh():
    return plsc.VectorSubcoreMesh(
        core_axis_name="c", subcore_axis_name="s", num_cores=NC, num_subcores=NS
    )


def _tc_precompute(x, pe, msg_w1, msg_b1, msgp_w1, msgp_b1):
    NB = 2000

    def body(x_r, pe_r, w1_r, b1_r, wp1_r, bp1_r, trec_r, tsend_r):
        xv = x_r[...]
        pev = pe_r[...]
        g = xv @ w1_r[0:H, :] + pev @ w1_r[H:2 * H, :] + b1_r[...]
        dr = pev @ wp1_r[H:2 * H, :] + bp1_r[...]
        cs = pev @ wp1_r[0:H, :]
        trec_r[...] = jnp.concatenate([g, dr], axis=1)
        tsend_r[...] = cs

    return pl.pallas_call(
        body,
        grid=(N // NB,),
        in_specs=[
            pl.BlockSpec((NB, H), lambda i: (i, 0)),
            pl.BlockSpec((NB, H), lambda i: (i, 0)),
            pl.BlockSpec((2 * H + 1, H), lambda i: (0, 0)),
            pl.BlockSpec((1, H), lambda i: (0, 0)),
            pl.BlockSpec((2 * H + 1, H), lambda i: (0, 0)),
            pl.BlockSpec((1, H), lambda i: (0, 0)),
        ],
        out_specs=[
            pl.BlockSpec((NB, TREC_D), lambda i: (i, 0)),
            pl.BlockSpec((NB, TSEND_D), lambda i: (i, 0)),
        ],
        out_shape=(
            jax.ShapeDtypeStruct((N, TREC_D), jnp.float32),
            jax.ShapeDtypeStruct((N, TSEND_D), jnp.float32),
        ),
    )(x, pe, msg_w1, msg_b1, msgp_w1, msgp_b1)


def _sc_gather(trec, tsend, posx, posy, posz, rec, send):
    @functools.partial(
        pl.kernel,
        out_type=(
            jax.ShapeDtypeStruct((SLAB, TREC_D), jnp.float32),
            jax.ShapeDtypeStruct((SLAB, PD), jnp.float32),
        ),
        mesh=_sc_mesh(),
        scratch_types=[
            pltpu.VMEM((GK,), jnp.int32),
            pltpu.VMEM((GK,), jnp.int32),
            pltpu.VMEM((GK, TREC_D), jnp.float32),
            pltpu.VMEM((GK, TSEND_D), jnp.float32),
            pltpu.VMEM((GK, PD), jnp.float32),
            pltpu.VMEM((GK,), jnp.int32),
            pltpu.VMEM((GK,), jnp.int32),
            pltpu.VMEM((GK, TREC_D), jnp.float32),
            pltpu.VMEM((GK, TSEND_D), jnp.float32),
            pltpu.VMEM((GK, PD), jnp.float32),
            pltpu.VMEM((GTAIL,), jnp.int32),
            pltpu.VMEM((GTAIL,), jnp.int32),
            pltpu.VMEM((N,), jnp.float32),
            pltpu.VMEM((N,), jnp.float32),
            pltpu.VMEM((N,), jnp.float32),
            pltpu.SemaphoreType.DMA,
            pltpu.SemaphoreType.DMA,
            pltpu.SemaphoreType.DMA,
            pltpu.SemaphoreType.DMA,
        ],
        compiler_params=pltpu.CompilerParams(needs_layout_passes=False),
    )
    def k(trec_h, tsend_h, posx_h, posy_h, posz_h, rec_h, send_h,
          grec_h, pose_h,
          idx_r0, idx_s0, buf_r0, buf_s0, pose0,
          idx_r1, idx_s1, buf_r1, buf_s1, pose1,
          idxt_r, idxt_s,
          posx_v, posy_v, posz_v,
          gsem0, gsem1, wsem0, wsem1):
        wid = lax.axis_index("s") * NC + lax.axis_index("c")
        base_w = wid * EPW
        pltpu.sync_copy(posx_h, posx_v)
        pltpu.sync_copy(posy_h, posy_v)
        pltpu.sync_copy(posz_h, posz_v)
        pos_tabs = (posx_v, posy_v, posz_v)

        zeros16 = jnp.zeros((16,), jnp.float32)
        iota16 = lax.iota(jnp.int32, 16)

        def load_idx(c, idx_r, idx_s, n):
            base = base_w + c * GK
            pltpu.sync_copy(rec_h.at[pl.ds(base, n)], idx_r)
            pltpu.sync_copy(send_h.at[pl.ds(base, n)], idx_s)

        def start_gather(idx_r, idx_s, buf_r, buf_s, gsem):
            pltpu.async_copy(trec_h.at[idx_r], buf_r, gsem)
            pltpu.async_copy(tsend_h.at[idx_s], buf_s, gsem)

        def wait_gather(idx_r, idx_s, buf_r, buf_s, gsem):
            pltpu.make_async_copy(trec_h.at[idx_r], buf_r, gsem).wait()
            pltpu.make_async_copy(tsend_h.at[idx_s], buf_s, gsem).wait()

        def do_adds(buf_r, buf_s, nrows):
            def row(e, carry):
                for kk in range(H // 16):
                    v = buf_s[e, pl.ds(kk * 16, 16)]
                    plsc.addupdate(buf_r.at[e, pl.ds(H + kk * 16, 16)], v)
                return carry
            lax.fori_loop(0, nrows, row, 0)

        def pose_fill(idx_s_ref, idx_r_ref, pose_ref, ngroups):
            for j in range(ngroups):
                ids = iota16 + j * 16
                si = idx_s_ref[pl.ds(j * 16, 16)]
                ri = idx_r_ref[pl.ds(j * 16, 16)]
                for c in range(3):
                    vs = plsc.load_gather(pos_tabs[c], [si])
                    vr = plsc.load_gather(pos_tabs[c], [ri])
                    plsc.store_scatter(
                        pose_ref, [ids, jnp.full((16,), c, jnp.int32)], vs)
                    plsc.store_scatter(
                        pose_ref, [ids, jnp.full((16,), 3 + c, jnp.int32)], vr)
                for c in (6, 7):
                    plsc.store_scatter(
                        pose_ref, [ids, jnp.full((16,), c, jnp.int32)], zeros16)

        def start_writes(buf_r, pose_b, c, wsem):
            base = base_w + c * GK
            pltpu.async_copy(buf_r, grec_h.at[pl.ds(base, GK)], wsem)
            pltpu.async_copy(pose_b, pose_h.at[pl.ds(base, GK)], wsem)

        def wait_writes(buf_r, pose_b, c, wsem):
            base = base_w + c * GK
            pltpu.make_async_copy(buf_r, grec_h.at[pl.ds(base, GK)], wsem).wait()
            pltpu.make_async_copy(pose_b, pose_h.at[pl.ds(base, GK)], wsem).wait()

        load_idx(0, idx_r0, idx_s0, GK)
        start_gather(idx_r0, idx_s0, buf_r0, buf_s0, gsem0)

        def pair(i, carry):
            c0 = 2 * i
            c1 = c0 + 1
            load_idx(c1, idx_r1, idx_s1, GK)

            @pl.when(i > 0)
            def _():
                wait_writes(buf_r1, pose1, c1 - 2, wsem1)

            start_gather(idx_r1, idx_s1, buf_r1, buf_s1, gsem1)
            pose_fill(idx_s0, idx_r0, pose0, GK // 16)
            wait_gather(idx_r0, idx_s0, buf_r0, buf_s0, gsem0)
            do_adds(buf_r0, buf_s0, GK)
            start_writes(buf_r0, pose0, c0, wsem0)

            @pl.when(i < GPAIRS - 1)
            def _():
                load_idx(c0 + 2, idx_r0, idx_s0, GK)
                wait_writes(buf_r0, pose0, c0, wsem0)
                start_gather(idx_r0, idx_s0, buf_r0, buf_s0, gsem0)

            pose_fill(idx_s1, idx_r1, pose1, GK // 16)
            wait_gather(idx_r1, idx_s1, buf_r1, buf_s1, gsem1)
            do_adds(buf_r1, buf_s1, GK)
            start_writes(buf_r1, pose1, c1, wsem1)
            return carry

        lax.fori_loop(0, GPAIRS, pair, 0)
        wait_writes(buf_r0, pose0, GFULL - 2, wsem0)
        wait_writes(buf_r1, pose1, GFULL - 1, wsem1)

        base = base_w + GFULL * GK
        pltpu.sync_copy(rec_h.at[pl.ds(base, GTAIL)], idxt_r)
        pltpu.sync_copy(send_h.at[pl.ds(base, GTAIL)], idxt_s)
        c1 = pltpu.async_copy(trec_h.at[idxt_r],
                              buf_r0.at[pl.ds(0, GTAIL)], gsem0)
        c2 = pltpu.async_copy(tsend_h.at[idxt_s],
                              buf_s0.at[pl.ds(0, GTAIL)], gsem0)
        pose_fill(idxt_s, idxt_r, pose0, GTAIL // 16)
        c1.wait()
        c2.wait()
        do_adds(buf_r0, buf_s0, GTAIL)
        pltpu.sync_copy(buf_r0.at[pl.ds(0, GTAIL)],
                        grec_h.at[pl.ds(base, GTAIL)])
        pltpu.sync_copy(pose0.at[pl.ds(0, GTAIL)],
                        pose_h.at[pl.ds(base, GTAIL)])

    return k(trec, tsend, posx, posy, posz, rec, send)


def _tc_edges(grec, pose, msg_w1, msgp_w1, msg_w2, msg_b2, msgp_w2, msgp_b2):
    B = 2560

    def body(grec_r, pose_r, w1_r, wp1_r, w2_r, b2_r, wp2_r, bp2_r,
             msg_r, msgp_r):
        g = grec_r[:, 0:H]
        s2 = grec_r[:, H:2 * H]
        ps = pose_r[:, 0:3]
        pr = pose_r[:, 3:6]
        d = ps - pr
        dist = jnp.sqrt(jnp.sum(d * d, axis=1, keepdims=True))
        w1d = w1_r[2 * H:2 * H + 1, :]
        wp1d = wp1_r[2 * H:2 * H + 1, :]
        h1 = jax.nn.silu(g + dist * w1d)
        msg_r[...] = jax.nn.silu(h1 @ w2_r[...] + b2_r[...])
        h1p = jnp.tanh(s2 + dist * wp1d)
        msgp_r[...] = jnp.tanh(h1p @ wp2_r[...] + bp2_r[...])

    return pl.pallas_call(
        body,
        grid=(SLAB // B,),
        in_specs=[
            pl.BlockSpec((B, TREC_D), lambda i: (i, 0)),
            pl.BlockSpec((B, PD), lambda i: (i, 0)),
            pl.BlockSpec((2 * H + 1, H), lambda i: (0, 0)),
            pl.BlockSpec((2 * H + 1, H), lambda i: (0, 0)),
            pl.BlockSpec((H, H), lambda i: (0, 0)),
            pl.BlockSpec((1, H), lambda i: (0, 0)),
            pl.BlockSpec((H, H), lambda i: (0, 0)),
            pl.BlockSpec((1, H), lambda i: (0, 0)),
        ],
        out_specs=[
            pl.BlockSpec((B, H), lambda i: (i, 0)),
            pl.BlockSpec((B, H), lambda i: (i, 0)),
        ],
        out_shape=(
            jax.ShapeDtypeStruct((SLAB, H), jnp.float32),
            jax.ShapeDtypeStruct((SLAB, H), jnp.float32),
        ),
    )(grec, pose, msg_w1, msgp_w1, msg_w2, msg_b2, msgp_w2, msgp_b2)


def _sc_scatter(msgs, msgps, rec, zeros):
    @functools.partial(
        pl.kernel,
        out_type=(
            jax.ShapeDtypeStruct((N, H), jnp.float32),
            jax.ShapeDtypeStruct((N, H), jnp.float32),
        ),
        mesh=_sc_mesh(),
        scratch_types=[
            pltpu.VMEM_SHARED((N, H), jnp.float32),
            pltpu.VMEM((SK,), jnp.int32),
            pltpu.VMEM((SK, H), jnp.float32),
            pltpu.VMEM((SK,), jnp.int32),
            pltpu.VMEM((SK, H), jnp.float32),
            pltpu.VMEM((STAIL,), jnp.int32),
            pltpu.VMEM((STAIL, H), jnp.float32),
            pltpu.SemaphoreType.DMA,
            pltpu.SemaphoreType.DMA,
        ],
    )
    def k(m0_h, m1_h, m2_h, m3_h, m4_h, p0_h, p1_h, p2_h, p3_h, p4_h,
          rec_h, zeros_h, aggr_h, aggrp_h,
          acc_s, idx0, mb0, idx1, mb1, idxt, mbt, lsem0, lsem1):
        msg_hs = (m0_h, m1_h, m2_h, m3_h, m4_h)
        msgp_hs = (p0_h, p1_h, p2_h, p3_h, p4_h)
        cid = lax.axis_index("c")
        sid = lax.axis_index("s")
        pltpu.sync_copy(zeros_h.at[pl.ds(sid * NROWS, NROWS)],
                        acc_s.at[pl.ds(sid * NROWS, NROWS)])

        @pl.when(sid == NS - 1)
        def _():
            pltpu.sync_copy(zeros_h.at[pl.ds(NS * NROWS, NREM)],
                            acc_s.at[pl.ds(NS * NROWS, NREM)])

        plsc.subcore_barrier()

        def run(src_hs):
            for k in range(NSLAB):
                src_h = src_hs[k]
                gbase = k * SLAB + sid * EPTS
                lbase = sid * EPTS

                def load(c, idx, mb, lsem):
                    pltpu.sync_copy(rec_h.at[pl.ds(gbase + c * SK, SK)], idx)
                    pltpu.async_copy(src_h.at[pl.ds(lbase + c * SK, SK)],
                                     mb, lsem)

                def wait_load(c, mb, lsem):
                    pltpu.make_async_copy(src_h.at[pl.ds(lbase + c * SK, SK)],
                                          mb, lsem).wait()

                load(0, idx0, mb0, lsem0)

                def pair(i, carry):
                    c0 = 2 * i
                    c1 = c0 + 1
                    load(c1, idx1, mb1, lsem1)
                    wait_load(c0, mb0, lsem0)
                    pltpu.sync_copy(mb0, acc_s.at[idx0], add=True)
                    load(c0 + 2, idx0, mb0, lsem0)
                    wait_load(c1, mb1, lsem1)
                    pltpu.sync_copy(mb1, acc_s.at[idx1], add=True)
                    return carry

                lax.fori_loop(0, SPAIRS, pair, 0)
                wait_load(SFULL - 1, mb0, lsem0)
                pltpu.sync_copy(mb0, acc_s.at[idx0], add=True)

                tb = SFULL * SK
                pltpu.sync_copy(rec_h.at[pl.ds(gbase + tb, STAIL)], idxt)
                pltpu.sync_copy(src_h.at[pl.ds(lbase + tb, STAIL)], mbt)
                pltpu.sync_copy(mbt, acc_s.at[idxt], add=True)

        @pl.when(cid == 0)
        def _():
            run(msg_hs)

        @pl.when(cid == 1)
        def _():
            run(msgp_hs)

        plsc.subcore_barrier()

        @pl.when(cid == 0)
        def _():
            pltpu.sync_copy(acc_s.at[pl.ds(sid * NROWS, NROWS)],
                            aggr_h.at[pl.ds(sid * NROWS, NROWS)])

        @pl.when(cid == 1)
        def _():
            pltpu.sync_copy(acc_s.at[pl.ds(sid * NROWS, NROWS)],
                            aggrp_h.at[pl.ds(sid * NROWS, NROWS)])

        @pl.when((sid == NS - 1) & (cid == 0))
        def _():
            pltpu.sync_copy(acc_s.at[pl.ds(NS * NROWS, NREM)],
                            aggr_h.at[pl.ds(NS * NROWS, NREM)])

        @pl.when((sid == NS - 1) & (cid == 1))
        def _():
            pltpu.sync_copy(acc_s.at[pl.ds(NS * NROWS, NREM)],
                            aggrp_h.at[pl.ds(NS * NROWS, NREM)])

    return k(*msgs, *msgps, rec, zeros)


def _tc_update(x, pe, aggr, aggrp, u1, ub1, u2, ub2, p1, pb1, p2, pb2):
    NB = 2000

    def body(x_r, pe_r, a_r, ap_r, u1_r, ub1_r, u2_r, ub2_r,
             p1_r, pb1_r, p2_r, pb2_r, out_r, outp_r):
        xv = x_r[...]
        pev = pe_r[...]
        t = (xv @ u1_r[0:H, :] + pev @ u1_r[H:2 * H, :]
             + a_r[...] @ u1_r[2 * H:3 * H, :] + ub1_r[...])
        out_r[...] = jax.nn.silu(t) @ u2_r[...] + ub2_r[...]
        tp = pev @ p1_r[0:H, :] + ap_r[...] @ p1_r[H:2 * H, :] + pb1_r[...]
        outp_r[...] = jnp.tanh(jnp.tanh(tp) @ p2_r[...] + pb2_r[...])

    return pl.pallas_call(
        body,
        grid=(N // NB,),
        in_specs=[
            pl.BlockSpec((NB, H), lambda i: (i, 0)),
            pl.BlockSpec((NB, H), lambda i: (i, 0)),
            pl.BlockSpec((NB, H), lambda i: (i, 0)),
            pl.BlockSpec((NB, H), lambda i: (i, 0)),
            pl.BlockSpec((3 * H, H), lambda i: (0, 0)),
            pl.BlockSpec((1, H), lambda i: (0, 0)),
            pl.BlockSpec((H, H), lambda i: (0, 0)),
            pl.BlockSpec((1, H), lambda i: (0, 0)),
            pl.BlockSpec((2 * H, H), lambda i: (0, 0)),
            pl.BlockSpec((1, H), lambda i: (0, 0)),
            pl.BlockSpec((H, H), lambda i: (0, 0)),
            pl.BlockSpec((1, H), lambda i: (0, 0)),
        ],
        out_specs=[
            pl.BlockSpec((NB, H), lambda i: (i, 0)),
            pl.BlockSpec((NB, H), lambda i: (i, 0)),
        ],
        out_shape=(
            jax.ShapeDtypeStruct((N, H), jnp.float32),
            jax.ShapeDtypeStruct((N, H), jnp.float32),
        ),
    )(x, pe, aggr, aggrp, u1, ub1, u2, ub2, p1, pb1, p2, pb2)


def kernel(x, pos, pe, edge_index, msg_w1, msg_b1, msg_w2, msg_b2,
           msgp_w1, msgp_b1, msgp_w2, msgp_b2, upd_w1, upd_b1, upd_w2,
           upd_b2, updp_w1, updp_b1, updp_w2, updp_b2):
    send = edge_index[0]
    rec = edge_index[1]

    b1 = msg_b1.reshape(1, H)
    b2 = msg_b2.reshape(1, H)
    bp1 = msgp_b1.reshape(1, H)
    bp2 = msgp_b2.reshape(1, H)
    ub1 = upd_b1.reshape(1, H)
    ub2 = upd_b2.reshape(1, H)
    pb1 = updp_b1.reshape(1, H)
    pb2 = updp_b2.reshape(1, H)

    posx = pos[:, 0]
    posy = pos[:, 1]
    posz = pos[:, 2]

    trec, tsend = _tc_precompute(x, pe, msg_w1, b1, msgp_w1, bp1)

    msgs = []
    msgps = []
    for k in range(NSLAB):
        rec_k = lax.slice_in_dim(rec, k * SLAB, (k + 1) * SLAB)
        send_k = lax.slice_in_dim(send, k * SLAB, (k + 1) * SLAB)
        grec, pose = _sc_gather(trec, tsend, posx, posy, posz, rec_k, send_k)
        m, mp = _tc_edges(grec, pose, msg_w1, msgp_w1, msg_w2, b2,
                          msgp_w2, bp2)
        msgs.append(m)
        msgps.append(mp)

    zeros = jnp.zeros((N, H), jnp.float32)
    aggr, aggrp = _sc_scatter(msgs, msgps, rec, zeros)
    return _tc_update(x, pe, aggr, aggrp, upd_w1, ub1, upd_w2, ub2,
                      updp_w1, pb1, updp_w2, pb2)


# speedup vs baseline: 4.9532x; 1.0446x over previous
import functools

import jax
import jax.numpy as jnp
from jax import lax
from jax.experimental import pallas as pl
from jax.experimental.pallas import tpu as pltpu
from jax.experimental.pallas import tpu_sc as plsc

N = 10000
E = 320000
H = 128
TREC_D = 2 * H
TSEND_D = H
PD = 8

NC = 2
NS = 16
NW = NC * NS

NSLAB = 5
SLAB = E // NSLAB

EPW = SLAB // NW
GK = 96
GFULL = EPW // GK
GPAIRS = GFULL // 2
GTAIL = EPW - GFULL * GK

EPTS = SLAB // NS
SK = 128
SFULL = EPTS // SK
SPAIRS = SFULL // 2
STAIL = EPTS - SFULL * SK
NROWS = 624
NREM = N - NS * NROWS


def _sc_mesh():
    return plsc.VectorSubcoreMesh(
        core_axis_name="c", subcore_axis_name="s", num_cores=NC, num_subcores=NS
    )


def _tc_precompute(x, pe, msg_w1, msg_b1, msgp_w1, msgp_b1):
    NB = 2000

    def body(x_r, pe_r, w1_r, b1_r, wp1_r, bp1_r, trec_r, tsend_r):
        xv = x_r[...]
        pev = pe_r[...]
        g = xv @ w1_r[0:H, :] + pev @ w1_r[H:2 * H, :] + b1_r[...]
        dr = pev @ wp1_r[H:2 * H, :] + bp1_r[...]
        cs = pev @ wp1_r[0:H, :]
        trec_r[...] = jnp.concatenate([g, dr], axis=1)
        tsend_r[...] = cs

    return pl.pallas_call(
        body,
        grid=(N // NB,),
        in_specs=[
            pl.BlockSpec((NB, H), lambda i: (i, 0)),
            pl.BlockSpec((NB, H), lambda i: (i, 0)),
            pl.BlockSpec((2 * H + 1, H), lambda i: (0, 0)),
            pl.BlockSpec((1, H), lambda i: (0, 0)),
            pl.BlockSpec((2 * H + 1, H), lambda i: (0, 0)),
            pl.BlockSpec((1, H), lambda i: (0, 0)),
        ],
        out_specs=[
            pl.BlockSpec((NB, TREC_D), lambda i: (i, 0)),
            pl.BlockSpec((NB, TSEND_D), lambda i: (i, 0)),
        ],
        out_shape=(
            jax.ShapeDtypeStruct((N, TREC_D), jnp.float32),
            jax.ShapeDtypeStruct((N, TSEND_D), jnp.float32),
        ),
    )(x, pe, msg_w1, msg_b1, msgp_w1, msgp_b1)


def _sc_gather(trec, tsend, posx, posy, posz, rec, send):
    @functools.partial(
        pl.kernel,
        out_type=(
            jax.ShapeDtypeStruct((SLAB, TREC_D), jnp.float32),
            jax.ShapeDtypeStruct((SLAB, PD), jnp.float32),
        ),
        mesh=_sc_mesh(),
        scratch_types=[
            pltpu.VMEM((GK,), jnp.int32),
            pltpu.VMEM((GK,), jnp.int32),
            pltpu.VMEM((GK, TREC_D), jnp.float32),
            pltpu.VMEM((GK, TSEND_D), jnp.float32),
            pltpu.VMEM((GK, PD), jnp.float32),
            pltpu.VMEM((GK,), jnp.int32),
            pltpu.VMEM((GK,), jnp.int32),
            pltpu.VMEM((GK, TREC_D), jnp.float32),
            pltpu.VMEM((GK, TSEND_D), jnp.float32),
            pltpu.VMEM((GK, PD), jnp.float32),
            pltpu.VMEM((GTAIL,), jnp.int32),
            pltpu.VMEM((GTAIL,), jnp.int32),
            pltpu.VMEM((N,), jnp.float32),
            pltpu.VMEM((N,), jnp.float32),
            pltpu.VMEM((N,), jnp.float32),
            pltpu.SemaphoreType.DMA,
            pltpu.SemaphoreType.DMA,
            pltpu.SemaphoreType.DMA,
            pltpu.SemaphoreType.DMA,
        ],
        compiler_params=pltpu.CompilerParams(needs_layout_passes=False),
    )
    def k(trec_h, tsend_h, posx_h, posy_h, posz_h, rec_h, send_h,
          grec_h, pose_h,
          idx_r0, idx_s0, buf_r0, buf_s0, pose0,
          idx_r1, idx_s1, buf_r1, buf_s1, pose1,
          idxt_r, idxt_s,
          posx_v, posy_v, posz_v,
          gsem0, gsem1, wsem0, wsem1):
        wid = lax.axis_index("s") * NC + lax.axis_index("c")
        base_w = wid * EPW
        pltpu.sync_copy(posx_h, posx_v)
        pltpu.sync_copy(posy_h, posy_v)
        pltpu.sync_copy(posz_h, posz_v)
        pos_tabs = (posx_v, posy_v, posz_v)

        zeros16 = jnp.zeros((16,), jnp.float32)
        iota16 = lax.iota(jnp.int32, 16)

        def load_idx(c, idx_r, idx_s, n):
            base = base_w + c * GK
            pltpu.sync_copy(rec_h.at[pl.ds(base, n)], idx_r)
            pltpu.sync_copy(send_h.at[pl.ds(base, n)], idx_s)

        def start_gather(idx_r, idx_s, buf_r, buf_s, gsem):
            pltpu.async_copy(trec_h.at[idx_r], buf_r, gsem)
            pltpu.async_copy(tsend_h.at[idx_s], buf_s, gsem)

        def wait_gather(idx_r, idx_s, buf_r, buf_s, gsem):
            pltpu.make_async_copy(trec_h.at[idx_r], buf_r, gsem).wait()
            pltpu.make_async_copy(tsend_h.at[idx_s], buf_s, gsem).wait()

        def do_adds(buf_r, buf_s, nrows):
            def row(e, carry):
                for kk in range(H // 16):
                    v = buf_s[e, pl.ds(kk * 16, 16)]
                    plsc.addupdate(buf_r.at[e, pl.ds(H + kk * 16, 16)], v)
                return carry
            lax.fori_loop(0, nrows, row, 0)

        def pose_fill(idx_s_ref, idx_r_ref, pose_ref, ngroups):
            for j in range(ngroups):
                ids = iota16 + j * 16
                si = idx_s_ref[pl.ds(j * 16, 16)]
                ri = idx_r_ref[pl.ds(j * 16, 16)]
                for c in range(3):
                    vs = plsc.load_gather(pos_tabs[c], [si])
                    vr = plsc.load_gather(pos_tabs[c], [ri])
                    plsc.store_scatter(
                        pose_ref, [ids, jnp.full((16,), c, jnp.int32)], vs)
                    plsc.store_scatter(
                        pose_ref, [ids, jnp.full((16,), 3 + c, jnp.int32)], vr)
                for c in (6, 7):
                    plsc.store_scatter(
                        pose_ref, [ids, jnp.full((16,), c, jnp.int32)], zeros16)

        def start_writes(buf_r, pose_b, c, wsem):
            base = base_w + c * GK
            pltpu.async_copy(buf_r, grec_h.at[pl.ds(base, GK)], wsem)
            pltpu.async_copy(pose_b, pose_h.at[pl.ds(base, GK)], wsem)

        def wait_writes(buf_r, pose_b, c, wsem):
            base = base_w + c * GK
            pltpu.make_async_copy(buf_r, grec_h.at[pl.ds(base, GK)], wsem).wait()
            pltpu.make_async_copy(pose_b, pose_h.at[pl.ds(base, GK)], wsem).wait()

        load_idx(0, idx_r0, idx_s0, GK)
        start_gather(idx_r0, idx_s0, buf_r0, buf_s0, gsem0)

        def pair(i, carry):
            c0 = 2 * i
            c1 = c0 + 1
            load_idx(c1, idx_r1, idx_s1, GK)

            @pl.when(i > 0)
            def _():
                wait_writes(buf_r1, pose1, c1 - 2, wsem1)

            start_gather(idx_r1, idx_s1, buf_r1, buf_s1, gsem1)
            pose_fill(idx_s0, idx_r0, pose0, GK // 16)
            wait_gather(idx_r0, idx_s0, buf_r0, buf_s0, gsem0)
            do_adds(buf_r0, buf_s0, GK)
            start_writes(buf_r0, pose0, c0, wsem0)

            @pl.when(i < GPAIRS - 1)
            def _():
                load_idx(c0 + 2, idx_r0, idx_s0, GK)
                wait_writes(buf_r0, pose0, c0, wsem0)
                start_gather(idx_r0, idx_s0, buf_r0, buf_s0, gsem0)

            pose_fill(idx_s1, idx_r1, pose1, GK // 16)
            wait_gather(idx_r1, idx_s1, buf_r1, buf_s1, gsem1)
            do_adds(buf_r1, buf_s1, GK)
            start_writes(buf_r1, pose1, c1, wsem1)
            return carry

        lax.fori_loop(0, GPAIRS, pair, 0)
        wait_writes(buf_r0, pose0, GFULL - 2, wsem0)
        wait_writes(buf_r1, pose1, GFULL - 1, wsem1)

        base = base_w + GFULL * GK
        pltpu.sync_copy(rec_h.at[pl.ds(base, GTAIL)], idxt_r)
        pltpu.sync_copy(send_h.at[pl.ds(base, GTAIL)], idxt_s)
        c1 = pltpu.async_copy(trec_h.at[idxt_r],
                              buf_r0.at[pl.ds(0, GTAIL)], gsem0)
        c2 = pltpu.async_copy(tsend_h.at[idxt_s],
                              buf_s0.at[pl.ds(0, GTAIL)], gsem0)
        pose_fill(idxt_s, idxt_r, pose0, GTAIL // 16)
        c1.wait()
        c2.wait()
        do_adds(buf_r0, buf_s0, GTAIL)
        pltpu.sync_copy(buf_r0.at[pl.ds(0, GTAIL)],
                        grec_h.at[pl.ds(base, GTAIL)])
        pltpu.sync_copy(pose0.at[pl.ds(0, GTAIL)],
                        pose_h.at[pl.ds(base, GTAIL)])

    return k(trec, tsend, posx, posy, posz, rec, send)


def _tc_edges(grec, pose, msg_w1, msgp_w1, msg_w2, msg_b2, msgp_w2, msgp_b2):
    B = 2560

    def body(grec_r, pose_r, w1_r, wp1_r, w2_r, b2_r, wp2_r, bp2_r,
             msg_r, msgp_r):
        g = grec_r[:, 0:H]
        s2 = grec_r[:, H:2 * H]
        ps = pose_r[:, 0:3]
        pr = pose_r[:, 3:6]
        d = ps - pr
        dist = jnp.sqrt(jnp.sum(d * d, axis=1, keepdims=True))
        w1d = w1_r[2 * H:2 * H + 1, :]
        wp1d = wp1_r[2 * H:2 * H + 1, :]
        h1 = jax.nn.silu(g + dist * w1d)
        msg_r[...] = jax.nn.silu(h1 @ w2_r[...] + b2_r[...])
        h1p = jnp.tanh(s2 + dist * wp1d)
        msgp_r[...] = jnp.tanh(h1p @ wp2_r[...] + bp2_r[...])

    return pl.pallas_call(
        body,
        grid=(SLAB // B,),
        in_specs=[
            pl.BlockSpec((B, TREC_D), lambda i: (i, 0)),
            pl.BlockSpec((B, PD), lambda i: (i, 0)),
            pl.BlockSpec((2 * H + 1, H), lambda i: (0, 0)),
            pl.BlockSpec((2 * H + 1, H), lambda i: (0, 0)),
            pl.BlockSpec((H, H), lambda i: (0, 0)),
            pl.BlockSpec((1, H), lambda i: (0, 0)),
            pl.BlockSpec((H, H), lambda i: (0, 0)),
            pl.BlockSpec((1, H), lambda i: (0, 0)),
        ],
        out_specs=[
            pl.BlockSpec((B, H), lambda i: (i, 0)),
            pl.BlockSpec((B, H), lambda i: (i, 0)),
        ],
        out_shape=(
            jax.ShapeDtypeStruct((SLAB, H), jnp.float32),
            jax.ShapeDtypeStruct((SLAB, H), jnp.float32),
        ),
    )(grec, pose, msg_w1, msgp_w1, msg_w2, msg_b2, msgp_w2, msgp_b2)


def _sc_scatter(msgs, msgps, slab_ids, rec, init0, init1):
    n = len(msgs)

    @functools.partial(
        pl.kernel,
        out_type=(
            jax.ShapeDtypeStruct((N, H), jnp.float32),
            jax.ShapeDtypeStruct((N, H), jnp.float32),
        ),
        mesh=_sc_mesh(),
        scratch_types=[
            pltpu.VMEM_SHARED((N, H), jnp.float32),
            pltpu.VMEM((SK,), jnp.int32),
            pltpu.VMEM((SK, H), jnp.float32),
            pltpu.VMEM((SK,), jnp.int32),
            pltpu.VMEM((SK, H), jnp.float32),
            pltpu.VMEM((STAIL,), jnp.int32),
            pltpu.VMEM((STAIL, H), jnp.float32),
            pltpu.SemaphoreType.DMA,
            pltpu.SemaphoreType.DMA,
        ],
    )
    def k(*refs):
        msg_hs = refs[0:n]
        msgp_hs = refs[n:2 * n]
        (rec_h, init0_h, init1_h, aggr_h, aggrp_h,
         acc_s, idx0, mb0, idx1, mb1, idxt, mbt, lsem0, lsem1) = refs[2 * n:]
        cid = lax.axis_index("c")
        sid = lax.axis_index("s")

        @pl.when(cid == 0)
        def _():
            pltpu.sync_copy(init0_h.at[pl.ds(sid * NROWS, NROWS)],
                            acc_s.at[pl.ds(sid * NROWS, NROWS)])

            @pl.when(sid == NS - 1)
            def _():
                pltpu.sync_copy(init0_h.at[pl.ds(NS * NROWS, NREM)],
                                acc_s.at[pl.ds(NS * NROWS, NREM)])

        @pl.when(cid == 1)
        def _():
            pltpu.sync_copy(init1_h.at[pl.ds(sid * NROWS, NROWS)],
                            acc_s.at[pl.ds(sid * NROWS, NROWS)])

            @pl.when(sid == NS - 1)
            def _():
                pltpu.sync_copy(init1_h.at[pl.ds(NS * NROWS, NREM)],
                                acc_s.at[pl.ds(NS * NROWS, NREM)])

        plsc.subcore_barrier()

        def run(src_hs):
            for j in range(n):
                src_h = src_hs[j]
                gbase = slab_ids[j] * SLAB + sid * EPTS
                lbase = sid * EPTS

                def load(c, idx, mb, lsem):
                    pltpu.sync_copy(rec_h.at[pl.ds(gbase + c * SK, SK)], idx)
                    pltpu.async_copy(src_h.at[pl.ds(lbase + c * SK, SK)],
                                     mb, lsem)

                def wait_load(c, mb, lsem):
                    pltpu.make_async_copy(src_h.at[pl.ds(lbase + c * SK, SK)],
                                          mb, lsem).wait()

                load(0, idx0, mb0, lsem0)

                def pair(i, carry):
                    c0 = 2 * i
                    c1 = c0 + 1
                    load(c1, idx1, mb1, lsem1)
                    wait_load(c0, mb0, lsem0)
                    pltpu.sync_copy(mb0, acc_s.at[idx0], add=True)
                    load(c0 + 2, idx0, mb0, lsem0)
                    wait_load(c1, mb1, lsem1)
                    pltpu.sync_copy(mb1, acc_s.at[idx1], add=True)
                    return carry

                lax.fori_loop(0, SPAIRS, pair, 0)
                wait_load(SFULL - 1, mb0, lsem0)
                pltpu.sync_copy(mb0, acc_s.at[idx0], add=True)

                tb = SFULL * SK
                pltpu.sync_copy(rec_h.at[pl.ds(gbase + tb, STAIL)], idxt)
                pltpu.sync_copy(src_h.at[pl.ds(lbase + tb, STAIL)], mbt)
                pltpu.sync_copy(mbt, acc_s.at[idxt], add=True)

        @pl.when(cid == 0)
        def _():
            run(msg_hs)

        @pl.when(cid == 1)
        def _():
            run(msgp_hs)

        plsc.subcore_barrier()

        @pl.when(cid == 0)
        def _():
            pltpu.sync_copy(acc_s.at[pl.ds(sid * NROWS, NROWS)],
                            aggr_h.at[pl.ds(sid * NROWS, NROWS)])

        @pl.when(cid == 1)
        def _():
            pltpu.sync_copy(acc_s.at[pl.ds(sid * NROWS, NROWS)],
                            aggrp_h.at[pl.ds(sid * NROWS, NROWS)])

        @pl.when((sid == NS - 1) & (cid == 0))
        def _():
            pltpu.sync_copy(acc_s.at[pl.ds(NS * NROWS, NREM)],
                            aggr_h.at[pl.ds(NS * NROWS, NREM)])

        @pl.when((sid == NS - 1) & (cid == 1))
        def _():
            pltpu.sync_copy(acc_s.at[pl.ds(NS * NROWS, NREM)],
                            aggrp_h.at[pl.ds(NS * NROWS, NREM)])

    return k(*msgs, *msgps, rec, init0, init1)


def _tc_update(x, pe, aggr, aggrp, u1, ub1, u2, ub2, p1, pb1, p2, pb2):
    NB = 2000

    def body(x_r, pe_r, a_r, ap_r, u1_r, ub1_r, u2_r, ub2_r,
             p1_r, pb1_r, p2_r, pb2_r, out_r, outp_r):
        xv = x_r[...]
        pev = pe_r[...]
        t = (xv @ u1_r[0:H, :] + pev @ u1_r[H:2 * H, :]
             + a_r[...] @ u1_r[2 * H:3 * H, :] + ub1_r[...])
        out_r[...] = jax.nn.silu(t) @ u2_r[...] + ub2_r[...]
        tp = pev @ p1_r[0:H, :] + ap_r[...] @ p1_r[H:2 * H, :] + pb1_r[...]
        outp_r[...] = jnp.tanh(jnp.tanh(tp) @ p2_r[...] + pb2_r[...])

    return pl.pallas_call(
        body,
        grid=(N // NB,),
        in_specs=[
            pl.BlockSpec((NB, H), lambda i: (i, 0)),
            pl.BlockSpec((NB, H), lambda i: (i, 0)),
            pl.BlockSpec((NB, H), lambda i: (i, 0)),
            pl.BlockSpec((NB, H), lambda i: (i, 0)),
            pl.BlockSpec((3 * H, H), lambda i: (0, 0)),
            pl.BlockSpec((1, H), lambda i: (0, 0)),
            pl.BlockSpec((H, H), lambda i: (0, 0)),
            pl.BlockSpec((1, H), lambda i: (0, 0)),
            pl.BlockSpec((2 * H, H), lambda i: (0, 0)),
            pl.BlockSpec((1, H), lambda i: (0, 0)),
            pl.BlockSpec((H, H), lambda i: (0, 0)),
            pl.BlockSpec((1, H), lambda i: (0, 0)),
        ],
        out_specs=[
            pl.BlockSpec((NB, H), lambda i: (i, 0)),
            pl.BlockSpec((NB, H), lambda i: (i, 0)),
        ],
        out_shape=(
            jax.ShapeDtypeStruct((N, H), jnp.float32),
            jax.ShapeDtypeStruct((N, H), jnp.float32),
        ),
    )(x, pe, aggr, aggrp, u1, ub1, u2, ub2, p1, pb1, p2, pb2)


def kernel(x, pos, pe, edge_index, msg_w1, msg_b1, msg_w2, msg_b2,
           msgp_w1, msgp_b1, msgp_w2, msgp_b2, upd_w1, upd_b1, upd_w2,
           upd_b2, updp_w1, updp_b1, updp_w2, updp_b2):
    send = edge_index[0]
    rec = edge_index[1]

    b1 = msg_b1.reshape(1, H)
    b2 = msg_b2.reshape(1, H)
    bp1 = msgp_b1.reshape(1, H)
    bp2 = msgp_b2.reshape(1, H)
    ub1 = upd_b1.reshape(1, H)
    ub2 = upd_b2.reshape(1, H)
    pb1 = updp_b1.reshape(1, H)
    pb2 = updp_b2.reshape(1, H)

    posx = pos[:, 0]
    posy = pos[:, 1]
    posz = pos[:, 2]

    trec, tsend = _tc_precompute(x, pe, msg_w1, b1, msgp_w1, bp1)

    msgs = []
    msgps = []
    for k in range(NSLAB):
        rec_k = lax.slice_in_dim(rec, k * SLAB, (k + 1) * SLAB)
        send_k = lax.slice_in_dim(send, k * SLAB, (k + 1) * SLAB)
        grec, pose = _sc_gather(trec, tsend, posx, posy, posz, rec_k, send_k)
        m, mp = _tc_edges(grec, pose, msg_w1, msgp_w1, msg_w2, b2,
                          msgp_w2, bp2)
        msgs.append(m)
        msgps.append(mp)

    zeros = jnp.zeros((N, H), jnp.float32)
    aggr1, aggrp1 = _sc_scatter(msgs[:3], msgps[:3], (0, 1, 2), rec,
                                zeros, zeros)
    aggr, aggrp = _sc_scatter(msgs[3:], msgps[3:], (3, 4), rec,
                              aggr1, aggrp1)
    return _tc_update(x, pe, aggr, aggrp, upd_w1, ub1, upd_w2, ub2,
                      updp_w1, pb1, updp_w2, pb2)


# speedup vs baseline: 5.6898x; 1.1487x over previous
import functools

import jax
import jax.numpy as jnp
from jax import lax
from jax.experimental import pallas as pl
from jax.experimental.pallas import tpu as pltpu
from jax.experimental.pallas import tpu_sc as plsc

N = 10000
E = 320000
H = 128
TREC_D = 2 * H
TSEND_D = H
PD = 8

NC = 2
NS = 16
NW = NC * NS

NSLAB = 5
SLAB = E // NSLAB

EPW = SLAB // NW
GK = 96
GFULL = EPW // GK
GPAIRS = GFULL // 2
GTAIL = EPW - GFULL * GK

EPTS = SLAB // NS
SK = 128
SFULL = EPTS // SK
SPAIRS = SFULL // 2
STAIL = EPTS - SFULL * SK
NROWS = 624
NREM = N - NS * NROWS


def _sc_mesh():
    return plsc.VectorSubcoreMesh(
        core_axis_name="c", subcore_axis_name="s", num_cores=NC, num_subcores=NS
    )


def _tc_precompute(x, pe, msg_w1, msg_b1, msgp_w1, msgp_b1):
    NB = 2000

    def body(x_r, pe_r, w1_r, b1_r, wp1_r, bp1_r, trec_r, tsend_r):
        xv = x_r[...]
        pev = pe_r[...]
        g = xv @ w1_r[0:H, :] + pev @ w1_r[H:2 * H, :] + b1_r[...]
        dr = pev @ wp1_r[H:2 * H, :] + bp1_r[...]
        cs = pev @ wp1_r[0:H, :]
        gi = jax.lax.bitcast_convert_type(g, jnp.int32) + 0x8000
        di = jax.lax.bitcast_convert_type(dr, jnp.int32) + 0x8000
        trec_r[...] = (gi & jnp.int32(-65536)) | jax.lax.shift_right_logical(
            di, 16)
        tsend_r[...] = cs

    return pl.pallas_call(
        body,
        grid=(N // NB,),
        in_specs=[
            pl.BlockSpec((NB, H), lambda i: (i, 0)),
            pl.BlockSpec((NB, H), lambda i: (i, 0)),
            pl.BlockSpec((2 * H + 1, H), lambda i: (0, 0)),
            pl.BlockSpec((1, H), lambda i: (0, 0)),
            pl.BlockSpec((2 * H + 1, H), lambda i: (0, 0)),
            pl.BlockSpec((1, H), lambda i: (0, 0)),
        ],
        out_specs=[
            pl.BlockSpec((NB, H), lambda i: (i, 0)),
            pl.BlockSpec((NB, TSEND_D), lambda i: (i, 0)),
        ],
        out_shape=(
            jax.ShapeDtypeStruct((N, H), jnp.int32),
            jax.ShapeDtypeStruct((N, TSEND_D), jnp.float32),
        ),
    )(x, pe, msg_w1, msg_b1, msgp_w1, msgp_b1)


def _sc_gather(trec, tsend, posx, posy, posz, rec, send):
    @functools.partial(
        pl.kernel,
        out_type=(
            jax.ShapeDtypeStruct((SLAB, H), jnp.int32),
            jax.ShapeDtypeStruct((SLAB, TSEND_D), jnp.float32),
            jax.ShapeDtypeStruct((SLAB, PD), jnp.float32),
        ),
        mesh=_sc_mesh(),
        scratch_types=[
            pltpu.VMEM((GK,), jnp.int32),
            pltpu.VMEM((GK,), jnp.int32),
            pltpu.VMEM((GK, H), jnp.int32),
            pltpu.VMEM((GK, TSEND_D), jnp.float32),
            pltpu.VMEM((GK, PD), jnp.float32),
            pltpu.VMEM((GK,), jnp.int32),
            pltpu.VMEM((GK,), jnp.int32),
            pltpu.VMEM((GK, H), jnp.int32),
            pltpu.VMEM((GK, TSEND_D), jnp.float32),
            pltpu.VMEM((GK, PD), jnp.float32),
            pltpu.VMEM((GTAIL,), jnp.int32),
            pltpu.VMEM((GTAIL,), jnp.int32),
            pltpu.VMEM((N,), jnp.float32),
            pltpu.VMEM((N,), jnp.float32),
            pltpu.VMEM((N,), jnp.float32),
            pltpu.SemaphoreType.DMA,
            pltpu.SemaphoreType.DMA,
            pltpu.SemaphoreType.DMA,
            pltpu.SemaphoreType.DMA,
        ],
        compiler_params=pltpu.CompilerParams(needs_layout_passes=False),
    )
    def k(trec_h, tsend_h, posx_h, posy_h, posz_h, rec_h, send_h,
          grec_h, gsend_h, pose_h,
          idx_r0, idx_s0, buf_r0, buf_s0, pose0,
          idx_r1, idx_s1, buf_r1, buf_s1, pose1,
          idxt_r, idxt_s,
          posx_v, posy_v, posz_v,
          gsem0, gsem1, wsem0, wsem1):
        wid = lax.axis_index("s") * NC + lax.axis_index("c")
        base_w = wid * EPW
        pltpu.sync_copy(posx_h, posx_v)
        pltpu.sync_copy(posy_h, posy_v)
        pltpu.sync_copy(posz_h, posz_v)
        pos_tabs = (posx_v, posy_v, posz_v)

        zeros16 = jnp.zeros((16,), jnp.float32)
        iota16 = lax.iota(jnp.int32, 16)

        def load_idx(c, idx_r, idx_s, n):
            base = base_w + c * GK
            pltpu.sync_copy(rec_h.at[pl.ds(base, n)], idx_r)
            pltpu.sync_copy(send_h.at[pl.ds(base, n)], idx_s)

        def start_gather(idx_r, idx_s, buf_r, buf_s, gsem):
            pltpu.async_copy(trec_h.at[idx_r], buf_r, gsem)
            pltpu.async_copy(tsend_h.at[idx_s], buf_s, gsem)

        def wait_gather(idx_r, idx_s, buf_r, buf_s, gsem):
            pltpu.make_async_copy(trec_h.at[idx_r], buf_r, gsem).wait()
            pltpu.make_async_copy(tsend_h.at[idx_s], buf_s, gsem).wait()

        def pose_fill(idx_s_ref, idx_r_ref, pose_ref, ngroups):
            for j in range(ngroups):
                ids = iota16 + j * 16
                si = idx_s_ref[pl.ds(j * 16, 16)]
                ri = idx_r_ref[pl.ds(j * 16, 16)]
                for c in range(3):
                    vs = plsc.load_gather(pos_tabs[c], [si])
                    vr = plsc.load_gather(pos_tabs[c], [ri])
                    plsc.store_scatter(
                        pose_ref, [ids, jnp.full((16,), c, jnp.int32)], vs)
                    plsc.store_scatter(
                        pose_ref, [ids, jnp.full((16,), 3 + c, jnp.int32)], vr)
                for c in (6, 7):
                    plsc.store_scatter(
                        pose_ref, [ids, jnp.full((16,), c, jnp.int32)], zeros16)

        def start_writes(buf_r, buf_s, pose_b, c, wsem):
            base = base_w + c * GK
            pltpu.async_copy(buf_r, grec_h.at[pl.ds(base, GK)], wsem)
            pltpu.async_copy(buf_s, gsend_h.at[pl.ds(base, GK)], wsem)
            pltpu.async_copy(pose_b, pose_h.at[pl.ds(base, GK)], wsem)

        def wait_writes(buf_r, buf_s, pose_b, c, wsem):
            base = base_w + c * GK
            pltpu.make_async_copy(buf_r, grec_h.at[pl.ds(base, GK)], wsem).wait()
            pltpu.make_async_copy(buf_s, gsend_h.at[pl.ds(base, GK)], wsem).wait()
            pltpu.make_async_copy(pose_b, pose_h.at[pl.ds(base, GK)], wsem).wait()

        load_idx(0, idx_r0, idx_s0, GK)
        start_gather(idx_r0, idx_s0, buf_r0, buf_s0, gsem0)

        def pair(i, carry):
            c0 = 2 * i
            c1 = c0 + 1
            load_idx(c1, idx_r1, idx_s1, GK)

            @pl.when(i > 0)
            def _():
                wait_writes(buf_r1, buf_s1, pose1, c1 - 2, wsem1)

            start_gather(idx_r1, idx_s1, buf_r1, buf_s1, gsem1)
            pose_fill(idx_s0, idx_r0, pose0, GK // 16)
            wait_gather(idx_r0, idx_s0, buf_r0, buf_s0, gsem0)
            start_writes(buf_r0, buf_s0, pose0, c0, wsem0)

            @pl.when(i < GPAIRS - 1)
            def _():
                load_idx(c0 + 2, idx_r0, idx_s0, GK)
                wait_writes(buf_r0, buf_s0, pose0, c0, wsem0)
                start_gather(idx_r0, idx_s0, buf_r0, buf_s0, gsem0)

            pose_fill(idx_s1, idx_r1, pose1, GK // 16)
            wait_gather(idx_r1, idx_s1, buf_r1, buf_s1, gsem1)
            start_writes(buf_r1, buf_s1, pose1, c1, wsem1)
            return carry

        lax.fori_loop(0, GPAIRS, pair, 0)
        wait_writes(buf_r0, buf_s0, pose0, GFULL - 2, wsem0)
        wait_writes(buf_r1, buf_s1, pose1, GFULL - 1, wsem1)

        base = base_w + GFULL * GK
        pltpu.sync_copy(rec_h.at[pl.ds(base, GTAIL)], idxt_r)
        pltpu.sync_copy(send_h.at[pl.ds(base, GTAIL)], idxt_s)
        c1 = pltpu.async_copy(trec_h.at[idxt_r],
                              buf_r0.at[pl.ds(0, GTAIL)], gsem0)
        c2 = pltpu.async_copy(tsend_h.at[idxt_s],
                              buf_s0.at[pl.ds(0, GTAIL)], gsem0)
        pose_fill(idxt_s, idxt_r, pose0, GTAIL // 16)
        c1.wait()
        c2.wait()
        pltpu.sync_copy(buf_r0.at[pl.ds(0, GTAIL)],
                        grec_h.at[pl.ds(base, GTAIL)])
        pltpu.sync_copy(buf_s0.at[pl.ds(0, GTAIL)],
                        gsend_h.at[pl.ds(base, GTAIL)])
        pltpu.sync_copy(pose0.at[pl.ds(0, GTAIL)],
                        pose_h.at[pl.ds(base, GTAIL)])

    return k(trec, tsend, posx, posy, posz, rec, send)


def _tc_edges(grec, gsend, pose, msg_w1, msgp_w1, msg_w2, msg_b2,
              msgp_w2, msgp_b2):
    B = 2560

    def body(grec_r, gsend_r, pose_r, w1_r, wp1_r, w2_r, b2_r, wp2_r, bp2_r,
             msg_r, msgp_r):
        packed = grec_r[...]
        g = jax.lax.bitcast_convert_type(
            packed & jnp.int32(-65536), jnp.float32)
        dr = jax.lax.bitcast_convert_type(
            jax.lax.shift_left(packed, 16), jnp.float32)
        cs = gsend_r[...]
        ps = pose_r[:, 0:3]
        pr = pose_r[:, 3:6]
        d = ps - pr
        dist = jnp.sqrt(jnp.sum(d * d, axis=1, keepdims=True))
        w1d = w1_r[2 * H:2 * H + 1, :]
        wp1d = wp1_r[2 * H:2 * H + 1, :]
        h1 = jax.nn.silu(g + dist * w1d)
        msg_r[...] = jax.nn.silu(h1 @ w2_r[...] + b2_r[...])
        h1p = jnp.tanh(cs + dr + dist * wp1d)
        msgp_r[...] = jnp.tanh(h1p @ wp2_r[...] + bp2_r[...])

    return pl.pallas_call(
        body,
        grid=(SLAB // B,),
        in_specs=[
            pl.BlockSpec((B, H), lambda i: (i, 0)),
            pl.BlockSpec((B, TSEND_D), lambda i: (i, 0)),
            pl.BlockSpec((B, PD), lambda i: (i, 0)),
            pl.BlockSpec((2 * H + 1, H), lambda i: (0, 0)),
            pl.BlockSpec((2 * H + 1, H), lambda i: (0, 0)),
            pl.BlockSpec((H, H), lambda i: (0, 0)),
            pl.BlockSpec((1, H), lambda i: (0, 0)),
            pl.BlockSpec((H, H), lambda i: (0, 0)),
            pl.BlockSpec((1, H), lambda i: (0, 0)),
        ],
        out_specs=[
            pl.BlockSpec((B, H), lambda i: (i, 0)),
            pl.BlockSpec((B, H), lambda i: (i, 0)),
        ],
        out_shape=(
            jax.ShapeDtypeStruct((SLAB, H), jnp.float32),
            jax.ShapeDtypeStruct((SLAB, H), jnp.float32),
        ),
    )(grec, gsend, pose, msg_w1, msgp_w1, msg_w2, msg_b2, msgp_w2, msgp_b2)


def _sc_scatter(msgs, msgps, slab_ids, rec, init0, init1):
    n = len(msgs)

    @functools.partial(
        pl.kernel,
        out_type=(
            jax.ShapeDtypeStruct((N, H), jnp.float32),
            jax.ShapeDtypeStruct((N, H), jnp.float32),
        ),
        mesh=_sc_mesh(),
        scratch_types=[
            pltpu.VMEM_SHARED((N, H), jnp.float32),
            pltpu.VMEM((SK,), jnp.int32),
            pltpu.VMEM((SK, H), jnp.float32),
            pltpu.VMEM((SK,), jnp.int32),
            pltpu.VMEM((SK, H), jnp.float32),
            pltpu.VMEM((STAIL,), jnp.int32),
            pltpu.VMEM((STAIL, H), jnp.float32),
            pltpu.SemaphoreType.DMA,
            pltpu.SemaphoreType.DMA,
        ],
    )
    def k(*refs):
        msg_hs = refs[0:n]
        msgp_hs = refs[n:2 * n]
        (rec_h, init0_h, init1_h, aggr_h, aggrp_h,
         acc_s, idx0, mb0, idx1, mb1, idxt, mbt, lsem0, lsem1) = refs[2 * n:]
        cid = lax.axis_index("c")
        sid = lax.axis_index("s")

        @pl.when(cid == 0)
        def _():
            pltpu.sync_copy(init0_h.at[pl.ds(sid * NROWS, NROWS)],
                            acc_s.at[pl.ds(sid * NROWS, NROWS)])

            @pl.when(sid == NS - 1)
            def _():
                pltpu.sync_copy(init0_h.at[pl.ds(NS * NROWS, NREM)],
                                acc_s.at[pl.ds(NS * NROWS, NREM)])

        @pl.when(cid == 1)
        def _():
            pltpu.sync_copy(init1_h.at[pl.ds(sid * NROWS, NROWS)],
                            acc_s.at[pl.ds(sid * NROWS, NROWS)])

            @pl.when(sid == NS - 1)
            def _():
                pltpu.sync_copy(init1_h.at[pl.ds(NS * NROWS, NREM)],
                                acc_s.at[pl.ds(NS * NROWS, NREM)])

        plsc.subcore_barrier()

        def run(src_hs):
            for j in range(n):
                src_h = src_hs[j]
                gbase = slab_ids[j] * SLAB + sid * EPTS
                lbase = sid * EPTS

                def load(c, idx, mb, lsem):
                    pltpu.sync_copy(rec_h.at[pl.ds(gbase + c * SK, SK)], idx)
                    pltpu.async_copy(src_h.at[pl.ds(lbase + c * SK, SK)],
                                     mb, lsem)

                def wait_load(c, mb, lsem):
                    pltpu.make_async_copy(src_h.at[pl.ds(lbase + c * SK, SK)],
                                          mb, lsem).wait()

                load(0, idx0, mb0, lsem0)

                def pair(i, carry):
                    c0 = 2 * i
                    c1 = c0 + 1
                    load(c1, idx1, mb1, lsem1)
                    wait_load(c0, mb0, lsem0)
                    pltpu.sync_copy(mb0, acc_s.at[idx0], add=True)
                    load(c0 + 2, idx0, mb0, lsem0)
                    wait_load(c1, mb1, lsem1)
                    pltpu.sync_copy(mb1, acc_s.at[idx1], add=True)
                    return carry

                lax.fori_loop(0, SPAIRS, pair, 0)
                wait_load(SFULL - 1, mb0, lsem0)
                pltpu.sync_copy(mb0, acc_s.at[idx0], add=True)

                tb = SFULL * SK
                pltpu.sync_copy(rec_h.at[pl.ds(gbase + tb, STAIL)], idxt)
                pltpu.sync_copy(src_h.at[pl.ds(lbase + tb, STAIL)], mbt)
                pltpu.sync_copy(mbt, acc_s.at[idxt], add=True)

        @pl.when(cid == 0)
        def _():
            run(msg_hs)

        @pl.when(cid == 1)
        def _():
            run(msgp_hs)

        plsc.subcore_barrier()

        @pl.when(cid == 0)
        def _():
            pltpu.sync_copy(acc_s.at[pl.ds(sid * NROWS, NROWS)],
                            aggr_h.at[pl.ds(sid * NROWS, NROWS)])

        @pl.when(cid == 1)
        def _():
            pltpu.sync_copy(acc_s.at[pl.ds(sid * NROWS, NROWS)],
                            aggrp_h.at[pl.ds(sid * NROWS, NROWS)])

        @pl.when((sid == NS - 1) & (cid == 0))
        def _():
            pltpu.sync_copy(acc_s.at[pl.ds(NS * NROWS, NREM)],
                            aggr_h.at[pl.ds(NS * NROWS, NREM)])

        @pl.when((sid == NS - 1) & (cid == 1))
        def _():
            pltpu.sync_copy(acc_s.at[pl.ds(NS * NROWS, NREM)],
                            aggrp_h.at[pl.ds(NS * NROWS, NREM)])

    return k(*msgs, *msgps, rec, init0, init1)


def _tc_update(x, pe, aggr, aggrp, u1, ub1, u2, ub2, p1, pb1, p2, pb2):
    NB = 2000

    def body(x_r, pe_r, a_r, ap_r, u1_r, ub1_r, u2_r, ub2_r,
             p1_r, pb1_r, p2_r, pb2_r, out_r, outp_r):
        xv = x_r[...]
        pev = pe_r[...]
        t = (xv @ u1_r[0:H, :] + pev @ u1_r[H:2 * H, :]
             + a_r[...] @ u1_r[2 * H:3 * H, :] + ub1_r[...])
        out_r[...] = jax.nn.silu(t) @ u2_r[...] + ub2_r[...]
        tp = pev @ p1_r[0:H, :] + ap_r[...] @ p1_r[H:2 * H, :] + pb1_r[...]
        outp_r[...] = jnp.tanh(jnp.tanh(tp) @ p2_r[...] + pb2_r[...])

    return pl.pallas_call(
        body,
        grid=(N // NB,),
        in_specs=[
            pl.BlockSpec((NB, H), lambda i: (i, 0)),
            pl.BlockSpec((NB, H), lambda i: (i, 0)),
            pl.BlockSpec((NB, H), lambda i: (i, 0)),
            pl.BlockSpec((NB, H), lambda i: (i, 0)),
            pl.BlockSpec((3 * H, H), lambda i: (0, 0)),
            pl.BlockSpec((1, H), lambda i: (0, 0)),
            pl.BlockSpec((H, H), lambda i: (0, 0)),
            pl.BlockSpec((1, H), lambda i: (0, 0)),
            pl.BlockSpec((2 * H, H), lambda i: (0, 0)),
            pl.BlockSpec((1, H), lambda i: (0, 0)),
            pl.BlockSpec((H, H), lambda i: (0, 0)),
            pl.BlockSpec((1, H), lambda i: (0, 0)),
        ],
        out_specs=[
            pl.BlockSpec((NB, H), lambda i: (i, 0)),
            pl.BlockSpec((NB, H), lambda i: (i, 0)),
        ],
        out_shape=(
            jax.ShapeDtypeStruct((N, H), jnp.float32),
            jax.ShapeDtypeStruct((N, H), jnp.float32),
        ),
    )(x, pe, aggr, aggrp, u1, ub1, u2, ub2, p1, pb1, p2, pb2)


def kernel(x, pos, pe, edge_index, msg_w1, msg_b1, msg_w2, msg_b2,
           msgp_w1, msgp_b1, msgp_w2, msgp_b2, upd_w1, upd_b1, upd_w2,
           upd_b2, updp_w1, updp_b1, updp_w2, updp_b2):
    send = edge_index[0]
    rec = edge_index[1]

    b1 = msg_b1.reshape(1, H)
    b2 = msg_b2.reshape(1, H)
    bp1 = msgp_b1.reshape(1, H)
    bp2 = msgp_b2.reshape(1, H)
    ub1 = upd_b1.reshape(1, H)
    ub2 = upd_b2.reshape(1, H)
    pb1 = updp_b1.reshape(1, H)
    pb2 = updp_b2.reshape(1, H)

    posx = pos[:, 0]
    posy = pos[:, 1]
    posz = pos[:, 2]

    trec, tsend = _tc_precompute(x, pe, msg_w1, b1, msgp_w1, bp1)

    msgs = []
    msgps = []
    for k in range(NSLAB):
        rec_k = lax.slice_in_dim(rec, k * SLAB, (k + 1) * SLAB)
        send_k = lax.slice_in_dim(send, k * SLAB, (k + 1) * SLAB)
        grec, gsend, pose = _sc_gather(trec, tsend, posx, posy, posz,
                                       rec_k, send_k)
        m, mp = _tc_edges(grec, gsend, pose, msg_w1, msgp_w1, msg_w2, b2,
                          msgp_w2, bp2)
        msgs.append(m)
        msgps.append(mp)

    zeros = jnp.zeros((N, H), jnp.float32)
    aggr1, aggrp1 = _sc_scatter(msgs[:3], msgps[:3], (0, 1, 2), rec,
                                zeros, zeros)
    aggr, aggrp = _sc_scatter(msgs[3:], msgps[3:], (3, 4), rec,
                              aggr1, aggrp1)
    return _tc_update(x, pe, aggr, aggrp, upd_w1, ub1, upd_w2, ub2,
                      updp_w1, pb1, updp_w2, pb2)


# speedup vs baseline: 5.7449x; 1.0097x over previous
import functools

import jax
import jax.numpy as jnp
from jax import lax
from jax.experimental import pallas as pl
from jax.experimental.pallas import tpu as pltpu
from jax.experimental.pallas import tpu_sc as plsc

N = 10000
E = 320000
H = 128
TREC_D = 2 * H
TSEND_D = H
PD = 8

NC = 2
NS = 16
NW = NC * NS

NSLAB = 5
SLAB = E // NSLAB

EPW = SLAB // NW
GK = 96
GFULL = EPW // GK
GPAIRS = GFULL // 2
GTAIL = EPW - GFULL * GK

EPTS = SLAB // NS
SK = 128
SFULL = EPTS // SK
SPAIRS = SFULL // 2
STAIL = EPTS - SFULL * SK
NROWS = 624
NREM = N - NS * NROWS


def _sc_mesh():
    return plsc.VectorSubcoreMesh(
        core_axis_name="c", subcore_axis_name="s", num_cores=NC, num_subcores=NS
    )


def _tc_precompute(x, pe, msg_w1, msg_b1, msgp_w1, msgp_b1):
    NB = 2000

    def body(x_r, pe_r, w1_r, b1_r, wp1_r, bp1_r, trec_r, tsend_r):
        xv = x_r[...]
        pev = pe_r[...]
        g = xv @ w1_r[0:H, :] + pev @ w1_r[H:2 * H, :] + b1_r[...]
        dr = pev @ wp1_r[H:2 * H, :] + bp1_r[...]
        cs = pev @ wp1_r[0:H, :]
        gi = jax.lax.bitcast_convert_type(g, jnp.int32) + 0x8000
        di = jax.lax.bitcast_convert_type(dr, jnp.int32) + 0x8000
        trec_r[...] = (gi & jnp.int32(-65536)) | jax.lax.shift_right_logical(
            di, 16)
        tsend_r[...] = cs

    return pl.pallas_call(
        body,
        grid=(N // NB,),
        in_specs=[
            pl.BlockSpec((NB, H), lambda i: (i, 0)),
            pl.BlockSpec((NB, H), lambda i: (i, 0)),
            pl.BlockSpec((2 * H + 1, H), lambda i: (0, 0)),
            pl.BlockSpec((1, H), lambda i: (0, 0)),
            pl.BlockSpec((2 * H + 1, H), lambda i: (0, 0)),
            pl.BlockSpec((1, H), lambda i: (0, 0)),
        ],
        out_specs=[
            pl.BlockSpec((NB, H), lambda i: (i, 0)),
            pl.BlockSpec((NB, TSEND_D), lambda i: (i, 0)),
        ],
        out_shape=(
            jax.ShapeDtypeStruct((N, H), jnp.int32),
            jax.ShapeDtypeStruct((N, TSEND_D), jnp.float32),
        ),
    )(x, pe, msg_w1, msg_b1, msgp_w1, msgp_b1)


def _sc_gather(trec, tsend, posx, posy, posz, rec, send):
    @functools.partial(
        pl.kernel,
        out_type=(
            jax.ShapeDtypeStruct((SLAB, H), jnp.int32),
            jax.ShapeDtypeStruct((SLAB, TSEND_D), jnp.float32),
            jax.ShapeDtypeStruct((SLAB, PD), jnp.float32),
        ),
        mesh=_sc_mesh(),
        scratch_types=[
            pltpu.VMEM((GK,), jnp.int32),
            pltpu.VMEM((GK,), jnp.int32),
            pltpu.VMEM((GK, H), jnp.int32),
            pltpu.VMEM((GK, TSEND_D), jnp.float32),
            pltpu.VMEM((GK, PD), jnp.float32),
            pltpu.VMEM((GK,), jnp.int32),
            pltpu.VMEM((GK,), jnp.int32),
            pltpu.VMEM((GK, H), jnp.int32),
            pltpu.VMEM((GK, TSEND_D), jnp.float32),
            pltpu.VMEM((GK, PD), jnp.float32),
            pltpu.VMEM((GTAIL,), jnp.int32),
            pltpu.VMEM((GTAIL,), jnp.int32),
            pltpu.VMEM((N,), jnp.float32),
            pltpu.VMEM((N,), jnp.float32),
            pltpu.VMEM((N,), jnp.float32),
            pltpu.SemaphoreType.DMA,
            pltpu.SemaphoreType.DMA,
            pltpu.SemaphoreType.DMA,
            pltpu.SemaphoreType.DMA,
        ],
        compiler_params=pltpu.CompilerParams(needs_layout_passes=False),
    )
    def k(trec_h, tsend_h, posx_h, posy_h, posz_h, rec_h, send_h,
          grec_h, gsend_h, pose_h,
          idx_r0, idx_s0, buf_r0, buf_s0, pose0,
          idx_r1, idx_s1, buf_r1, buf_s1, pose1,
          idxt_r, idxt_s,
          posx_v, posy_v, posz_v,
          gsem0, gsem1, wsem0, wsem1):
        wid = lax.axis_index("s") * NC + lax.axis_index("c")
        base_w = wid * EPW
        ptab0 = pltpu.async_copy(posx_h, posx_v, wsem1)
        ptab1 = pltpu.async_copy(posy_h, posy_v, wsem1)
        ptab2 = pltpu.async_copy(posz_h, posz_v, wsem1)
        pos_tabs = (posx_v, posy_v, posz_v)

        zeros16 = jnp.zeros((16,), jnp.float32)
        iota16 = lax.iota(jnp.int32, 16)

        def load_idx(c, idx_r, idx_s, n):
            base = base_w + c * GK
            pltpu.sync_copy(rec_h.at[pl.ds(base, n)], idx_r)
            pltpu.sync_copy(send_h.at[pl.ds(base, n)], idx_s)

        def start_gather(idx_r, idx_s, buf_r, buf_s, gsem):
            pltpu.async_copy(trec_h.at[idx_r], buf_r, gsem)
            pltpu.async_copy(tsend_h.at[idx_s], buf_s, gsem)

        def wait_gather(idx_r, idx_s, buf_r, buf_s, gsem):
            pltpu.make_async_copy(trec_h.at[idx_r], buf_r, gsem).wait()
            pltpu.make_async_copy(tsend_h.at[idx_s], buf_s, gsem).wait()

        def pose_fill(idx_s_ref, idx_r_ref, pose_ref, ngroups):
            for j in range(ngroups):
                ids = iota16 + j * 16
                si = idx_s_ref[pl.ds(j * 16, 16)]
                ri = idx_r_ref[pl.ds(j * 16, 16)]
                for c in range(3):
                    vs = plsc.load_gather(pos_tabs[c], [si])
                    vr = plsc.load_gather(pos_tabs[c], [ri])
                    plsc.store_scatter(
                        pose_ref, [ids, jnp.full((16,), c, jnp.int32)], vs)
                    plsc.store_scatter(
                        pose_ref, [ids, jnp.full((16,), 3 + c, jnp.int32)], vr)
                for c in (6, 7):
                    plsc.store_scatter(
                        pose_ref, [ids, jnp.full((16,), c, jnp.int32)], zeros16)

        def start_writes(buf_r, buf_s, pose_b, c, wsem):
            base = base_w + c * GK
            pltpu.async_copy(buf_r, grec_h.at[pl.ds(base, GK)], wsem)
            pltpu.async_copy(buf_s, gsend_h.at[pl.ds(base, GK)], wsem)
            pltpu.async_copy(pose_b, pose_h.at[pl.ds(base, GK)], wsem)

        def wait_writes(buf_r, buf_s, pose_b, c, wsem):
            base = base_w + c * GK
            pltpu.make_async_copy(buf_r, grec_h.at[pl.ds(base, GK)], wsem).wait()
            pltpu.make_async_copy(buf_s, gsend_h.at[pl.ds(base, GK)], wsem).wait()
            pltpu.make_async_copy(pose_b, pose_h.at[pl.ds(base, GK)], wsem).wait()

        load_idx(0, idx_r0, idx_s0, GK)
        start_gather(idx_r0, idx_s0, buf_r0, buf_s0, gsem0)
        ptab0.wait()
        ptab1.wait()
        ptab2.wait()

        def pair(i, carry):
            c0 = 2 * i
            c1 = c0 + 1
            load_idx(c1, idx_r1, idx_s1, GK)

            @pl.when(i > 0)
            def _():
                wait_writes(buf_r1, buf_s1, pose1, c1 - 2, wsem1)

            start_gather(idx_r1, idx_s1, buf_r1, buf_s1, gsem1)
            pose_fill(idx_s0, idx_r0, pose0, GK // 16)
            wait_gather(idx_r0, idx_s0, buf_r0, buf_s0, gsem0)
            start_writes(buf_r0, buf_s0, pose0, c0, wsem0)

            @pl.when(i < GPAIRS - 1)
            def _():
                load_idx(c0 + 2, idx_r0, idx_s0, GK)
                wait_writes(buf_r0, buf_s0, pose0, c0, wsem0)
                start_gather(idx_r0, idx_s0, buf_r0, buf_s0, gsem0)

            pose_fill(idx_s1, idx_r1, pose1, GK // 16)
            wait_gather(idx_r1, idx_s1, buf_r1, buf_s1, gsem1)
            start_writes(buf_r1, buf_s1, pose1, c1, wsem1)
            return carry

        lax.fori_loop(0, GPAIRS, pair, 0)
        wait_writes(buf_r0, buf_s0, pose0, GFULL - 2, wsem0)
        wait_writes(buf_r1, buf_s1, pose1, GFULL - 1, wsem1)

        base = base_w + GFULL * GK
        pltpu.sync_copy(rec_h.at[pl.ds(base, GTAIL)], idxt_r)
        pltpu.sync_copy(send_h.at[pl.ds(base, GTAIL)], idxt_s)
        c1 = pltpu.async_copy(trec_h.at[idxt_r],
                              buf_r0.at[pl.ds(0, GTAIL)], gsem0)
        c2 = pltpu.async_copy(tsend_h.at[idxt_s],
                              buf_s0.at[pl.ds(0, GTAIL)], gsem0)
        pose_fill(idxt_s, idxt_r, pose0, GTAIL // 16)
        c1.wait()
        c2.wait()
        pltpu.sync_copy(buf_r0.at[pl.ds(0, GTAIL)],
                        grec_h.at[pl.ds(base, GTAIL)])
        pltpu.sync_copy(buf_s0.at[pl.ds(0, GTAIL)],
                        gsend_h.at[pl.ds(base, GTAIL)])
        pltpu.sync_copy(pose0.at[pl.ds(0, GTAIL)],
                        pose_h.at[pl.ds(base, GTAIL)])

    return k(trec, tsend, posx, posy, posz, rec, send)


def _tc_edges(grec, gsend, pose, msg_w1, msgp_w1, msg_w2, msg_b2,
              msgp_w2, msgp_b2):
    B = 2560

    def body(grec_r, gsend_r, pose_r, w1_r, wp1_r, w2_r, b2_r, wp2_r, bp2_r,
             msg_r, msgp_r):
        packed = grec_r[...]
        g = jax.lax.bitcast_convert_type(
            packed & jnp.int32(-65536), jnp.float32)
        dr = jax.lax.bitcast_convert_type(
            jax.lax.shift_left(packed, 16), jnp.float32)
        cs = gsend_r[...]
        ps = pose_r[:, 0:3]
        pr = pose_r[:, 3:6]
        d = ps - pr
        dist = jnp.sqrt(jnp.sum(d * d, axis=1, keepdims=True))
        w1d = w1_r[2 * H:2 * H + 1, :]
        wp1d = wp1_r[2 * H:2 * H + 1, :]
        h1 = jax.nn.silu(g + dist * w1d)
        msg_r[...] = jax.nn.silu(h1 @ w2_r[...] + b2_r[...])
        h1p = jnp.tanh(cs + dr + dist * wp1d)
        msgp_r[...] = jnp.tanh(h1p @ wp2_r[...] + bp2_r[...])

    return pl.pallas_call(
        body,
        grid=(SLAB // B,),
        in_specs=[
            pl.BlockSpec((B, H), lambda i: (i, 0)),
            pl.BlockSpec((B, TSEND_D), lambda i: (i, 0)),
            pl.BlockSpec((B, PD), lambda i: (i, 0)),
            pl.BlockSpec((2 * H + 1, H), lambda i: (0, 0)),
            pl.BlockSpec((2 * H + 1, H), lambda i: (0, 0)),
            pl.BlockSpec((H, H), lambda i: (0, 0)),
            pl.BlockSpec((1, H), lambda i: (0, 0)),
            pl.BlockSpec((H, H), lambda i: (0, 0)),
            pl.BlockSpec((1, H), lambda i: (0, 0)),
        ],
        out_specs=[
            pl.BlockSpec((B, H), lambda i: (i, 0)),
            pl.BlockSpec((B, H), lambda i: (i, 0)),
        ],
        out_shape=(
            jax.ShapeDtypeStruct((SLAB, H), jnp.float32),
            jax.ShapeDtypeStruct((SLAB, H), jnp.float32),
        ),
    )(grec, gsend, pose, msg_w1, msgp_w1, msg_w2, msg_b2, msgp_w2, msgp_b2)


def _sc_scatter(msgs, msgps, slab_ids, rec, init0, init1):
    n = len(msgs)

    @functools.partial(
        pl.kernel,
        out_type=(
            jax.ShapeDtypeStruct((N, H), jnp.float32),
            jax.ShapeDtypeStruct((N, H), jnp.float32),
        ),
        mesh=_sc_mesh(),
        scratch_types=[
            pltpu.VMEM_SHARED((N, H), jnp.float32),
            pltpu.VMEM((SK,), jnp.int32),
            pltpu.VMEM((SK, H), jnp.float32),
            pltpu.VMEM((SK,), jnp.int32),
            pltpu.VMEM((SK, H), jnp.float32),
            pltpu.VMEM((STAIL,), jnp.int32),
            pltpu.VMEM((STAIL, H), jnp.float32),
            pltpu.SemaphoreType.DMA,
            pltpu.SemaphoreType.DMA,
        ],
    )
    def k(*refs):
        msg_hs = refs[0:n]
        msgp_hs = refs[n:2 * n]
        (rec_h, init0_h, init1_h, aggr_h, aggrp_h,
         acc_s, idx0, mb0, idx1, mb1, idxt, mbt, lsem0, lsem1) = refs[2 * n:]
        cid = lax.axis_index("c")
        sid = lax.axis_index("s")

        @pl.when(cid == 0)
        def _():
            pltpu.sync_copy(init0_h.at[pl.ds(sid * NROWS, NROWS)],
                            acc_s.at[pl.ds(sid * NROWS, NROWS)])

            @pl.when(sid == NS - 1)
            def _():
                pltpu.sync_copy(init0_h.at[pl.ds(NS * NROWS, NREM)],
                                acc_s.at[pl.ds(NS * NROWS, NREM)])

        @pl.when(cid == 1)
        def _():
            pltpu.sync_copy(init1_h.at[pl.ds(sid * NROWS, NROWS)],
                            acc_s.at[pl.ds(sid * NROWS, NROWS)])

            @pl.when(sid == NS - 1)
            def _():
                pltpu.sync_copy(init1_h.at[pl.ds(NS * NROWS, NREM)],
                                acc_s.at[pl.ds(NS * NROWS, NREM)])

        plsc.subcore_barrier()

        def run(src_hs):
            for j in range(n):
                src_h = src_hs[j]
                gbase = slab_ids[j] * SLAB + sid * EPTS
                lbase = sid * EPTS

                def load(c, idx, mb, lsem):
                    pltpu.sync_copy(rec_h.at[pl.ds(gbase + c * SK, SK)], idx)
                    pltpu.async_copy(src_h.at[pl.ds(lbase + c * SK, SK)],
                                     mb, lsem)

                def wait_load(c, mb, lsem):
                    pltpu.make_async_copy(src_h.at[pl.ds(lbase + c * SK, SK)],
                                          mb, lsem).wait()

                load(0, idx0, mb0, lsem0)

                def pair(i, carry):
                    c0 = 2 * i
                    c1 = c0 + 1
                    load(c1, idx1, mb1, lsem1)
                    wait_load(c0, mb0, lsem0)
                    pltpu.sync_copy(mb0, acc_s.at[idx0], add=True)
                    load(c0 + 2, idx0, mb0, lsem0)
                    wait_load(c1, mb1, lsem1)
                    pltpu.sync_copy(mb1, acc_s.at[idx1], add=True)
                    return carry

                lax.fori_loop(0, SPAIRS, pair, 0)
                wait_load(SFULL - 1, mb0, lsem0)
                pltpu.sync_copy(mb0, acc_s.at[idx0], add=True)

                tb = SFULL * SK
                pltpu.sync_copy(rec_h.at[pl.ds(gbase + tb, STAIL)], idxt)
                pltpu.sync_copy(src_h.at[pl.ds(lbase + tb, STAIL)], mbt)
                pltpu.sync_copy(mbt, acc_s.at[idxt], add=True)

        @pl.when(cid == 0)
        def _():
            run(msg_hs)

        @pl.when(cid == 1)
        def _():
            run(msgp_hs)

        plsc.subcore_barrier()

        @pl.when(cid == 0)
        def _():
            pltpu.sync_copy(acc_s.at[pl.ds(sid * NROWS, NROWS)],
                            aggr_h.at[pl.ds(sid * NROWS, NROWS)])

        @pl.when(cid == 1)
        def _():
            pltpu.sync_copy(acc_s.at[pl.ds(sid * NROWS, NROWS)],
                            aggrp_h.at[pl.ds(sid * NROWS, NROWS)])

        @pl.when((sid == NS - 1) & (cid == 0))
        def _():
            pltpu.sync_copy(acc_s.at[pl.ds(NS * NROWS, NREM)],
                            aggr_h.at[pl.ds(NS * NROWS, NREM)])

        @pl.when((sid == NS - 1) & (cid == 1))
        def _():
            pltpu.sync_copy(acc_s.at[pl.ds(NS * NROWS, NREM)],
                            aggrp_h.at[pl.ds(NS * NROWS, NREM)])

    return k(*msgs, *msgps, rec, init0, init1)


def _tc_update(x, pe, aggr, aggrp, u1, ub1, u2, ub2, p1, pb1, p2, pb2):
    NB = 2000

    def body(x_r, pe_r, a_r, ap_r, u1_r, ub1_r, u2_r, ub2_r,
             p1_r, pb1_r, p2_r, pb2_r, out_r, outp_r):
        xv = x_r[...]
        pev = pe_r[...]
        t = (xv @ u1_r[0:H, :] + pev @ u1_r[H:2 * H, :]
             + a_r[...] @ u1_r[2 * H:3 * H, :] + ub1_r[...])
        out_r[...] = jax.nn.silu(t) @ u2_r[...] + ub2_r[...]
        tp = pev @ p1_r[0:H, :] + ap_r[...] @ p1_r[H:2 * H, :] + pb1_r[...]
        outp_r[...] = jnp.tanh(jnp.tanh(tp) @ p2_r[...] + pb2_r[...])

    return pl.pallas_call(
        body,
        grid=(N // NB,),
        in_specs=[
            pl.BlockSpec((NB, H), lambda i: (i, 0)),
            pl.BlockSpec((NB, H), lambda i: (i, 0)),
            pl.BlockSpec((NB, H), lambda i: (i, 0)),
            pl.BlockSpec((NB, H), lambda i: (i, 0)),
            pl.BlockSpec((3 * H, H), lambda i: (0, 0)),
            pl.BlockSpec((1, H), lambda i: (0, 0)),
            pl.BlockSpec((H, H), lambda i: (0, 0)),
            pl.BlockSpec((1, H), lambda i: (0, 0)),
            pl.BlockSpec((2 * H, H), lambda i: (0, 0)),
            pl.BlockSpec((1, H), lambda i: (0, 0)),
            pl.BlockSpec((H, H), lambda i: (0, 0)),
            pl.BlockSpec((1, H), lambda i: (0, 0)),
        ],
        out_specs=[
            pl.BlockSpec((NB, H), lambda i: (i, 0)),
            pl.BlockSpec((NB, H), lambda i: (i, 0)),
        ],
        out_shape=(
            jax.ShapeDtypeStruct((N, H), jnp.float32),
            jax.ShapeDtypeStruct((N, H), jnp.float32),
        ),
    )(x, pe, aggr, aggrp, u1, ub1, u2, ub2, p1, pb1, p2, pb2)


def kernel(x, pos, pe, edge_index, msg_w1, msg_b1, msg_w2, msg_b2,
           msgp_w1, msgp_b1, msgp_w2, msgp_b2, upd_w1, upd_b1, upd_w2,
           upd_b2, updp_w1, updp_b1, updp_w2, updp_b2):
    send = edge_index[0]
    rec = edge_index[1]

    b1 = msg_b1.reshape(1, H)
    b2 = msg_b2.reshape(1, H)
    bp1 = msgp_b1.reshape(1, H)
    bp2 = msgp_b2.reshape(1, H)
    ub1 = upd_b1.reshape(1, H)
    ub2 = upd_b2.reshape(1, H)
    pb1 = updp_b1.reshape(1, H)
    pb2 = updp_b2.reshape(1, H)

    posx = pos[:, 0]
    posy = pos[:, 1]
    posz = pos[:, 2]

    trec, tsend = _tc_precompute(x, pe, msg_w1, b1, msgp_w1, bp1)

    msgs = []
    msgps = []
    for k in range(NSLAB):
        rec_k = lax.slice_in_dim(rec, k * SLAB, (k + 1) * SLAB)
        send_k = lax.slice_in_dim(send, k * SLAB, (k + 1) * SLAB)
        grec, gsend, pose = _sc_gather(trec, tsend, posx, posy, posz,
                                       rec_k, send_k)
        m, mp = _tc_edges(grec, gsend, pose, msg_w1, msgp_w1, msg_w2, b2,
                          msgp_w2, bp2)
        msgs.append(m)
        msgps.append(mp)

    zeros = jnp.zeros((N, H), jnp.float32)
    aggr1, aggrp1 = _sc_scatter(msgs[:3], msgps[:3], (0, 1, 2), rec,
                                zeros, zeros)
    aggr, aggrp = _sc_scatter(msgs[3:], msgps[3:], (3, 4), rec,
                              aggr1, aggrp1)
    return _tc_update(x, pe, aggr, aggrp, upd_w1, ub1, upd_w2, ub2,
                      updp_w1, pb1, updp_w2, pb2)


# speedup vs baseline: 5.9311x; 1.0324x over previous
import functools

import jax
import jax.numpy as jnp
from jax import lax
from jax.experimental import pallas as pl
from jax.experimental.pallas import tpu as pltpu
from jax.experimental.pallas import tpu_sc as plsc

N = 10000
E = 320000
H = 128
TREC_D = 2 * H
TSEND_D = H
PD = 8

NC = 2
NS = 16
NW = NC * NS

NSLAB = 5
SLAB = E // NSLAB

EPW = SLAB // NW
GK = 96
GFULL = EPW // GK
GPAIRS = GFULL // 2
GTAIL = EPW - GFULL * GK

EPTS = SLAB // NS
SK = 128
SFULL = EPTS // SK
SPAIRS = SFULL // 2
STAIL = EPTS - SFULL * SK
NROWS = 624
NREM = N - NS * NROWS


def _sc_mesh():
    return plsc.VectorSubcoreMesh(
        core_axis_name="c", subcore_axis_name="s", num_cores=NC, num_subcores=NS
    )


def _tc_precompute(x, pe, msg_w1, msg_b1, msgp_w1, msgp_b1):
    NB = 2000

    def body(x_r, pe_r, w1_r, b1_r, wp1_r, bp1_r, trec_r, tsend_r):
        xv = x_r[...]
        pev = pe_r[...]
        g = xv @ w1_r[0:H, :] + pev @ w1_r[H:2 * H, :] + b1_r[...]
        dr = pev @ wp1_r[H:2 * H, :] + bp1_r[...]
        cs = pev @ wp1_r[0:H, :]
        gi = jax.lax.bitcast_convert_type(g, jnp.int32) + 0x8000
        di = jax.lax.bitcast_convert_type(dr, jnp.int32) + 0x8000
        trec_r[...] = (gi & jnp.int32(-65536)) | jax.lax.shift_right_logical(
            di, 16)
        tsend_r[...] = cs

    return pl.pallas_call(
        body,
        grid=(N // NB,),
        in_specs=[
            pl.BlockSpec((NB, H), lambda i: (i, 0)),
            pl.BlockSpec((NB, H), lambda i: (i, 0)),
            pl.BlockSpec((2 * H + 1, H), lambda i: (0, 0)),
            pl.BlockSpec((1, H), lambda i: (0, 0)),
            pl.BlockSpec((2 * H + 1, H), lambda i: (0, 0)),
            pl.BlockSpec((1, H), lambda i: (0, 0)),
        ],
        out_specs=[
            pl.BlockSpec((NB, H), lambda i: (i, 0)),
            pl.BlockSpec((NB, TSEND_D), lambda i: (i, 0)),
        ],
        out_shape=(
            jax.ShapeDtypeStruct((N, H), jnp.int32),
            jax.ShapeDtypeStruct((N, TSEND_D), jnp.float32),
        ),
    )(x, pe, msg_w1, msg_b1, msgp_w1, msgp_b1)


def _sc_gather(trec, tsend, posx, posy, posz, rec, send):
    @functools.partial(
        pl.kernel,
        out_type=(
            jax.ShapeDtypeStruct((SLAB, H), jnp.int32),
            jax.ShapeDtypeStruct((SLAB, TSEND_D), jnp.float32),
            jax.ShapeDtypeStruct((SLAB, PD), jnp.float32),
        ),
        mesh=_sc_mesh(),
        scratch_types=[
            pltpu.VMEM((GK,), jnp.int32),
            pltpu.VMEM((GK,), jnp.int32),
            pltpu.VMEM((GK, H), jnp.int32),
            pltpu.VMEM((GK, TSEND_D), jnp.float32),
            pltpu.VMEM((GK, PD), jnp.float32),
            pltpu.VMEM((GK,), jnp.int32),
            pltpu.VMEM((GK,), jnp.int32),
            pltpu.VMEM((GK, H), jnp.int32),
            pltpu.VMEM((GK, TSEND_D), jnp.float32),
            pltpu.VMEM((GK, PD), jnp.float32),
            pltpu.VMEM((GTAIL,), jnp.int32),
            pltpu.VMEM((GTAIL,), jnp.int32),
            pltpu.VMEM((N,), jnp.float32),
            pltpu.VMEM((N,), jnp.float32),
            pltpu.VMEM((N,), jnp.float32),
            pltpu.SemaphoreType.DMA,
            pltpu.SemaphoreType.DMA,
            pltpu.SemaphoreType.DMA,
            pltpu.SemaphoreType.DMA,
        ],
        compiler_params=pltpu.CompilerParams(needs_layout_passes=False),
    )
    def k(trec_h, tsend_h, posx_h, posy_h, posz_h, rec_h, send_h,
          grec_h, gsend_h, pose_h,
          idx_r0, idx_s0, buf_r0, buf_s0, pose0,
          idx_r1, idx_s1, buf_r1, buf_s1, pose1,
          idxt_r, idxt_s,
          posx_v, posy_v, posz_v,
          gsem0, gsem1, wsem0, wsem1):
        wid = lax.axis_index("s") * NC + lax.axis_index("c")
        base_w = wid * EPW
        ptab0 = pltpu.async_copy(posx_h, posx_v, wsem1)
        ptab1 = pltpu.async_copy(posy_h, posy_v, wsem1)
        ptab2 = pltpu.async_copy(posz_h, posz_v, wsem1)
        pos_tabs = (posx_v, posy_v, posz_v)

        zeros16 = jnp.zeros((16,), jnp.float32)
        iota16 = lax.iota(jnp.int32, 16)

        def load_idx(c, idx_r, idx_s, n):
            base = base_w + c * GK
            pltpu.sync_copy(rec_h.at[pl.ds(base, n)], idx_r)
            pltpu.sync_copy(send_h.at[pl.ds(base, n)], idx_s)

        def start_gather(idx_r, idx_s, buf_r, buf_s, gsem):
            pltpu.async_copy(trec_h.at[idx_r], buf_r, gsem)
            pltpu.async_copy(tsend_h.at[idx_s], buf_s, gsem)

        def wait_gather(idx_r, idx_s, buf_r, buf_s, gsem):
            pltpu.make_async_copy(trec_h.at[idx_r], buf_r, gsem).wait()
            pltpu.make_async_copy(tsend_h.at[idx_s], buf_s, gsem).wait()

        def pose_fill(idx_s_ref, idx_r_ref, pose_ref, ngroups):
            for j in range(ngroups):
                ids = iota16 + j * 16
                si = idx_s_ref[pl.ds(j * 16, 16)]
                ri = idx_r_ref[pl.ds(j * 16, 16)]
                for c in range(3):
                    vs = plsc.load_gather(pos_tabs[c], [si])
                    vr = plsc.load_gather(pos_tabs[c], [ri])
                    plsc.store_scatter(
                        pose_ref, [ids, jnp.full((16,), c, jnp.int32)], vs)
                    plsc.store_scatter(
                        pose_ref, [ids, jnp.full((16,), 3 + c, jnp.int32)], vr)
                for c in (6, 7):
                    plsc.store_scatter(
                        pose_ref, [ids, jnp.full((16,), c, jnp.int32)], zeros16)

        def start_writes(buf_r, buf_s, pose_b, c, wsem):
            base = base_w + c * GK
            pltpu.async_copy(buf_r, grec_h.at[pl.ds(base, GK)], wsem)
            pltpu.async_copy(buf_s, gsend_h.at[pl.ds(base, GK)], wsem)
            pltpu.async_copy(pose_b, pose_h.at[pl.ds(base, GK)], wsem)

        def wait_writes(buf_r, buf_s, pose_b, c, wsem):
            base = base_w + c * GK
            pltpu.make_async_copy(buf_r, grec_h.at[pl.ds(base, GK)], wsem).wait()
            pltpu.make_async_copy(buf_s, gsend_h.at[pl.ds(base, GK)], wsem).wait()
            pltpu.make_async_copy(pose_b, pose_h.at[pl.ds(base, GK)], wsem).wait()

        load_idx(0, idx_r0, idx_s0, GK)
        start_gather(idx_r0, idx_s0, buf_r0, buf_s0, gsem0)
        ptab0.wait()
        ptab1.wait()
        ptab2.wait()

        def pair(i, carry):
            c0 = 2 * i
            c1 = c0 + 1
            load_idx(c1, idx_r1, idx_s1, GK)

            @pl.when(i > 0)
            def _():
                wait_writes(buf_r1, buf_s1, pose1, c1 - 2, wsem1)

            start_gather(idx_r1, idx_s1, buf_r1, buf_s1, gsem1)
            pose_fill(idx_s0, idx_r0, pose0, GK // 16)
            wait_gather(idx_r0, idx_s0, buf_r0, buf_s0, gsem0)
            start_writes(buf_r0, buf_s0, pose0, c0, wsem0)

            @pl.when(i < GPAIRS - 1)
            def _():
                load_idx(c0 + 2, idx_r0, idx_s0, GK)
                wait_writes(buf_r0, buf_s0, pose0, c0, wsem0)
                start_gather(idx_r0, idx_s0, buf_r0, buf_s0, gsem0)

            pose_fill(idx_s1, idx_r1, pose1, GK // 16)
            wait_gather(idx_r1, idx_s1, buf_r1, buf_s1, gsem1)
            start_writes(buf_r1, buf_s1, pose1, c1, wsem1)
            return carry

        lax.fori_loop(0, GPAIRS, pair, 0)
        wait_writes(buf_r0, buf_s0, pose0, GFULL - 2, wsem0)
        wait_writes(buf_r1, buf_s1, pose1, GFULL - 1, wsem1)

        base = base_w + GFULL * GK
        pltpu.sync_copy(rec_h.at[pl.ds(base, GTAIL)], idxt_r)
        pltpu.sync_copy(send_h.at[pl.ds(base, GTAIL)], idxt_s)
        c1 = pltpu.async_copy(trec_h.at[idxt_r],
                              buf_r0.at[pl.ds(0, GTAIL)], gsem0)
        c2 = pltpu.async_copy(tsend_h.at[idxt_s],
                              buf_s0.at[pl.ds(0, GTAIL)], gsem0)
        pose_fill(idxt_s, idxt_r, pose0, GTAIL // 16)
        c1.wait()
        c2.wait()
        pltpu.sync_copy(buf_r0.at[pl.ds(0, GTAIL)],
                        grec_h.at[pl.ds(base, GTAIL)])
        pltpu.sync_copy(buf_s0.at[pl.ds(0, GTAIL)],
                        gsend_h.at[pl.ds(base, GTAIL)])
        pltpu.sync_copy(pose0.at[pl.ds(0, GTAIL)],
                        pose_h.at[pl.ds(base, GTAIL)])

    return k(trec, tsend, posx, posy, posz, rec, send)


def _tc_edges(grec, gsend, pose, msg_w1, msgp_w1, msg_w2, msg_b2,
              msgp_w2, msgp_b2):
    B = 2560

    def body(grec_r, gsend_r, pose_r, w1_r, wp1_r, w2_r, b2_r, wp2_r, bp2_r,
             msg_r, msgp_r):
        packed = grec_r[...]
        g = jax.lax.bitcast_convert_type(
            packed & jnp.int32(-65536), jnp.float32)
        dr = jax.lax.bitcast_convert_type(
            jax.lax.shift_left(packed, 16), jnp.float32)
        cs = gsend_r[...]
        ps = pose_r[:, 0:3]
        pr = pose_r[:, 3:6]
        d = ps - pr
        dist = jnp.sqrt(jnp.sum(d * d, axis=1, keepdims=True))
        w1d = w1_r[2 * H:2 * H + 1, :]
        wp1d = wp1_r[2 * H:2 * H + 1, :]
        h1 = jax.nn.silu(g + dist * w1d)
        msg_r[...] = jax.nn.silu(h1 @ w2_r[...] + b2_r[...])
        h1p = jnp.tanh(cs + dr + dist * wp1d)
        msgp_r[...] = jnp.tanh(h1p @ wp2_r[...] + bp2_r[...])

    return pl.pallas_call(
        body,
        grid=(SLAB // B,),
        in_specs=[
            pl.BlockSpec((B, H), lambda i: (i, 0)),
            pl.BlockSpec((B, TSEND_D), lambda i: (i, 0)),
            pl.BlockSpec((B, PD), lambda i: (i, 0)),
            pl.BlockSpec((2 * H + 1, H), lambda i: (0, 0)),
            pl.BlockSpec((2 * H + 1, H), lambda i: (0, 0)),
            pl.BlockSpec((H, H), lambda i: (0, 0)),
            pl.BlockSpec((1, H), lambda i: (0, 0)),
            pl.BlockSpec((H, H), lambda i: (0, 0)),
            pl.BlockSpec((1, H), lambda i: (0, 0)),
        ],
        out_specs=[
            pl.BlockSpec((B, H), lambda i: (i, 0)),
            pl.BlockSpec((B, H), lambda i: (i, 0)),
        ],
        out_shape=(
            jax.ShapeDtypeStruct((SLAB, H), jnp.float32),
            jax.ShapeDtypeStruct((SLAB, H), jnp.float32),
        ),
    )(grec, gsend, pose, msg_w1, msgp_w1, msg_w2, msg_b2, msgp_w2, msgp_b2)


def _sc_scatter(msgs, msgps, slab_ids, rec, init0, init1):
    n = len(msgs)

    @functools.partial(
        pl.kernel,
        out_type=(
            jax.ShapeDtypeStruct((N, H), jnp.float32),
            jax.ShapeDtypeStruct((N, H), jnp.float32),
        ),
        mesh=_sc_mesh(),
        scratch_types=[
            pltpu.VMEM_SHARED((N, H), jnp.float32),
            pltpu.VMEM((SK,), jnp.int32),
            pltpu.VMEM((SK, H), jnp.float32),
            pltpu.VMEM((SK,), jnp.int32),
            pltpu.VMEM((SK, H), jnp.float32),
            pltpu.VMEM((STAIL,), jnp.int32),
            pltpu.VMEM((STAIL, H), jnp.float32),
            pltpu.SemaphoreType.DMA,
            pltpu.SemaphoreType.DMA,
        ],
    )
    def k(*refs):
        msg_hs = refs[0:n]
        msgp_hs = refs[n:2 * n]
        (rec_h, init0_h, init1_h, aggr_h, aggrp_h,
         acc_s, idx0, mb0, idx1, mb1, idxt, mbt, lsem0, lsem1) = refs[2 * n:]
        cid = lax.axis_index("c")
        sid = lax.axis_index("s")

        @pl.when(cid == 0)
        def _():
            pltpu.sync_copy(init0_h.at[pl.ds(sid * NROWS, NROWS)],
                            acc_s.at[pl.ds(sid * NROWS, NROWS)])

            @pl.when(sid == NS - 1)
            def _():
                pltpu.sync_copy(init0_h.at[pl.ds(NS * NROWS, NREM)],
                                acc_s.at[pl.ds(NS * NROWS, NREM)])

        @pl.when(cid == 1)
        def _():
            pltpu.sync_copy(init1_h.at[pl.ds(sid * NROWS, NROWS)],
                            acc_s.at[pl.ds(sid * NROWS, NROWS)])

            @pl.when(sid == NS - 1)
            def _():
                pltpu.sync_copy(init1_h.at[pl.ds(NS * NROWS, NREM)],
                                acc_s.at[pl.ds(NS * NROWS, NREM)])

        plsc.subcore_barrier()

        def run(src_hs):
            for j in range(n):
                src_h = src_hs[j]
                gbase = slab_ids[j] * SLAB + sid * EPTS
                lbase = sid * EPTS

                def load(c, idx, mb, lsem):
                    pltpu.async_copy(rec_h.at[pl.ds(gbase + c * SK, SK)],
                                     idx, lsem)
                    pltpu.async_copy(src_h.at[pl.ds(lbase + c * SK, SK)],
                                     mb, lsem)

                def wait_load(c, idx, mb, lsem):
                    pltpu.make_async_copy(rec_h.at[pl.ds(gbase + c * SK, SK)],
                                          idx, lsem).wait()
                    pltpu.make_async_copy(src_h.at[pl.ds(lbase + c * SK, SK)],
                                          mb, lsem).wait()

                load(0, idx0, mb0, lsem0)

                def pair(i, carry):
                    c0 = 2 * i
                    c1 = c0 + 1
                    load(c1, idx1, mb1, lsem1)
                    wait_load(c0, idx0, mb0, lsem0)
                    pltpu.sync_copy(mb0, acc_s.at[idx0], add=True)
                    load(c0 + 2, idx0, mb0, lsem0)
                    wait_load(c1, idx1, mb1, lsem1)
                    pltpu.sync_copy(mb1, acc_s.at[idx1], add=True)
                    return carry

                lax.fori_loop(0, SPAIRS, pair, 0)
                wait_load(SFULL - 1, idx0, mb0, lsem0)
                pltpu.sync_copy(mb0, acc_s.at[idx0], add=True)

                tb = SFULL * SK
                pltpu.sync_copy(rec_h.at[pl.ds(gbase + tb, STAIL)], idxt)
                pltpu.sync_copy(src_h.at[pl.ds(lbase + tb, STAIL)], mbt)
                pltpu.sync_copy(mbt, acc_s.at[idxt], add=True)

        @pl.when(cid == 0)
        def _():
            run(msg_hs)

        @pl.when(cid == 1)
        def _():
            run(msgp_hs)

        plsc.subcore_barrier()

        @pl.when(cid == 0)
        def _():
            pltpu.sync_copy(acc_s.at[pl.ds(sid * NROWS, NROWS)],
                            aggr_h.at[pl.ds(sid * NROWS, NROWS)])

        @pl.when(cid == 1)
        def _():
            pltpu.sync_copy(acc_s.at[pl.ds(sid * NROWS, NROWS)],
                            aggrp_h.at[pl.ds(sid * NROWS, NROWS)])

        @pl.when((sid == NS - 1) & (cid == 0))
        def _():
            pltpu.sync_copy(acc_s.at[pl.ds(NS * NROWS, NREM)],
                            aggr_h.at[pl.ds(NS * NROWS, NREM)])

        @pl.when((sid == NS - 1) & (cid == 1))
        def _():
            pltpu.sync_copy(acc_s.at[pl.ds(NS * NROWS, NREM)],
                            aggrp_h.at[pl.ds(NS * NROWS, NREM)])

    return k(*msgs, *msgps, rec, init0, init1)


def _tc_update(x, pe, aggr, aggrp, u1, ub1, u2, ub2, p1, pb1, p2, pb2):
    NB = 2000

    def body(x_r, pe_r, a_r, ap_r, u1_r, ub1_r, u2_r, ub2_r,
             p1_r, pb1_r, p2_r, pb2_r, out_r, outp_r):
        xv = x_r[...]
        pev = pe_r[...]
        t = (xv @ u1_r[0:H, :] + pev @ u1_r[H:2 * H, :]
             + a_r[...] @ u1_r[2 * H:3 * H, :] + ub1_r[...])
        out_r[...] = jax.nn.silu(t) @ u2_r[...] + ub2_r[...]
        tp = pev @ p1_r[0:H, :] + ap_r[...] @ p1_r[H:2 * H, :] + pb1_r[...]
        outp_r[...] = jnp.tanh(jnp.tanh(tp) @ p2_r[...] + pb2_r[...])

    return pl.pallas_call(
        body,
        grid=(N // NB,),
        in_specs=[
            pl.BlockSpec((NB, H), lambda i: (i, 0)),
            pl.BlockSpec((NB, H), lambda i: (i, 0)),
            pl.BlockSpec((NB, H), lambda i: (i, 0)),
            pl.BlockSpec((NB, H), lambda i: (i, 0)),
            pl.BlockSpec((3 * H, H), lambda i: (0, 0)),
            pl.BlockSpec((1, H), lambda i: (0, 0)),
            pl.BlockSpec((H, H), lambda i: (0, 0)),
            pl.BlockSpec((1, H), lambda i: (0, 0)),
            pl.BlockSpec((2 * H, H), lambda i: (0, 0)),
            pl.BlockSpec((1, H), lambda i: (0, 0)),
            pl.BlockSpec((H, H), lambda i: (0, 0)),
            pl.BlockSpec((1, H), lambda i: (0, 0)),
        ],
        out_specs=[
            pl.BlockSpec((NB, H), lambda i: (i, 0)),
            pl.BlockSpec((NB, H), lambda i: (i, 0)),
        ],
        out_shape=(
            jax.ShapeDtypeStruct((N, H), jnp.float32),
            jax.ShapeDtypeStruct((N, H), jnp.float32),
        ),
    )(x, pe, aggr, aggrp, u1, ub1, u2, ub2, p1, pb1, p2, pb2)


def kernel(x, pos, pe, edge_index, msg_w1, msg_b1, msg_w2, msg_b2,
           msgp_w1, msgp_b1, msgp_w2, msgp_b2, upd_w1, upd_b1, upd_w2,
           upd_b2, updp_w1, updp_b1, updp_w2, updp_b2):
    send = edge_index[0]
    rec = edge_index[1]

    b1 = msg_b1.reshape(1, H)
    b2 = msg_b2.reshape(1, H)
    bp1 = msgp_b1.reshape(1, H)
    bp2 = msgp_b2.reshape(1, H)
    ub1 = upd_b1.reshape(1, H)
    ub2 = upd_b2.reshape(1, H)
    pb1 = updp_b1.reshape(1, H)
    pb2 = updp_b2.reshape(1, H)

    posx = pos[:, 0]
    posy = pos[:, 1]
    posz = pos[:, 2]

    trec, tsend = _tc_precompute(x, pe, msg_w1, b1, msgp_w1, bp1)

    msgs = []
    msgps = []
    for k in range(NSLAB):
        rec_k = lax.slice_in_dim(rec, k * SLAB, (k + 1) * SLAB)
        send_k = lax.slice_in_dim(send, k * SLAB, (k + 1) * SLAB)
        grec, gsend, pose = _sc_gather(trec, tsend, posx, posy, posz,
                                       rec_k, send_k)
        m, mp = _tc_edges(grec, gsend, pose, msg_w1, msgp_w1, msg_w2, b2,
                          msgp_w2, bp2)
        msgs.append(m)
        msgps.append(mp)

    zeros = jnp.zeros((N, H), jnp.float32)
    aggr1, aggrp1 = _sc_scatter(msgs[:3], msgps[:3], (0, 1, 2), rec,
                                zeros, zeros)
    aggr, aggrp = _sc_scatter(msgs[3:], msgps[3:], (3, 4), rec,
                              aggr1, aggrp1)
    return _tc_update(x, pe, aggr, aggrp, upd_w1, ub1, upd_w2, ub2,
                      updp_w1, pb1, updp_w2, pb2)
